# Initial kernel scaffold; baseline (speedup 1.0000x reference)
import jax
import jax.numpy as jnp
from jax.experimental import pallas as pl


def kernel(x, edge_source, edge_target, edge_attr, global_fea, node_batch, W_x, b_x, W_e, b_e, conv_W, conv_att, conv_bias, bn1_g, bn1_b, bn_g, bn_b, ca_W1, ca_b1, ca_W2, ca_b2, post_W, post_b, out_W, out_b):
    raise NotImplementedError("write your pallas kernel here")


# speedup vs baseline: 9.2204x; 9.2204x over previous
import functools

import jax
import jax.numpy as jnp
from jax import lax
from jax.experimental import pallas as pl
from jax.experimental.pallas import tpu as pltpu

_N, _E, _G = 10000, 160000, 128
_TILE = 3200
_NT = _E // _TILE

_INTERPRET = False


def _sp(x):
    return jnp.maximum(x, 0.0) + jnp.log(1.0 + jnp.exp(-jnp.abs(x)))


def _passA_body(xi_ref, xj_ref, ea_ref, wt_ref, wb_ref, ai_ref, aj_ref,
                msel_ref, gb_ref, spr_ref, consts_ref, acc_ref):
    t = pl.program_id(0)
    eb = jnp.dot(ea_ref[...], wb_ref[...], preferred_element_type=jnp.float32)
    zi = jnp.dot(xi_ref[...], wt_ref[...], preferred_element_type=jnp.float32) + eb
    zj = jnp.dot(xj_ref[...], wt_ref[...], preferred_element_type=jnp.float32) + eb
    oi = _sp(zi)
    oj = _sp(zj)
    prod = oi * ai_ref[...] + oj * aj_ref[...]
    a_raw = jnp.dot(prod, msel_ref[...], preferred_element_type=jnp.float32)
    spr = _sp(a_raw)
    spr_ref[...] = spr

    pad = jnp.zeros((124,), jnp.float32)
    s1 = jnp.concatenate([jnp.sum(spr, 0), pad])[None, :]
    s2 = jnp.concatenate([jnp.sum(spr * spr, 0), pad])[None, :]
    mn = jnp.concatenate([jnp.min(spr, 0), jnp.full((124,), jnp.inf, jnp.float32)])[None, :]
    mx = jnp.concatenate([jnp.max(spr, 0), jnp.full((124,), -jnp.inf, jnp.float32)])[None, :]

    @pl.when(t == 0)
    def _():
        acc_ref[0:1, :] = s1
        acc_ref[1:2, :] = s2
        acc_ref[2:3, :] = mn
        acc_ref[3:4, :] = mx

    @pl.when(t > 0)
    def _():
        acc_ref[0:1, :] = acc_ref[0:1, :] + s1
        acc_ref[1:2, :] = acc_ref[1:2, :] + s2
        acc_ref[2:3, :] = jnp.minimum(acc_ref[2:3, :], mn)
        acc_ref[3:4, :] = jnp.maximum(acc_ref[3:4, :], mx)

    @pl.when(t == _NT - 1)
    def _():
        g = gb_ref[0:1, :]
        b = gb_ref[1:2, :]
        mu = acc_ref[0:1, :] / _E
        var = acc_ref[1:2, :] / _E - mu * mu
        inv = lax.rsqrt(var + 1e-5)
        A = g * inv
        B = b - g * mu * inv
        y_hi = jnp.maximum(A * acc_ref[3:4, :] + B, A * acc_ref[2:3, :] + B)
        c = 1.0 / (1.0 + jnp.exp(y_hi))
        consts_ref[0:1, :] = A
        consts_ref[1:2, :] = B
        consts_ref[2:3, :] = c


def _passA(xi, xj, ea, wt, wb, ai_flat, aj_flat, msel, gb):
    return pl.pallas_call(
        _passA_body,
        grid=(_NT,),
        in_specs=[
            pl.BlockSpec((_TILE, 64), lambda t: (t, 0)),
            pl.BlockSpec((_TILE, 64), lambda t: (t, 0)),
            pl.BlockSpec((_TILE, 64), lambda t: (t, 0)),
            pl.BlockSpec((64, 256), lambda t: (0, 0)),
            pl.BlockSpec((64, 256), lambda t: (0, 0)),
            pl.BlockSpec((1, 256), lambda t: (0, 0)),
            pl.BlockSpec((1, 256), lambda t: (0, 0)),
            pl.BlockSpec((256, 4), lambda t: (0, 0)),
            pl.BlockSpec((2, 128), lambda t: (0, 0)),
        ],
        out_specs=[
            pl.BlockSpec((_TILE, 4), lambda t: (t, 0)),
            pl.BlockSpec((4, 128), lambda t: (0, 0)),
        ],
        out_shape=[
            jax.ShapeDtypeStruct((_E, 4), jnp.float32),
            jax.ShapeDtypeStruct((4, 128), jnp.float32),
        ],
        scratch_shapes=[pltpu.VMEM((4, 128), jnp.float32)],
        interpret=_INTERPRET,
    )(xi, xj, ea, wt, wb, ai_flat, aj_flat, msel, gb)


def _passC_body(xj_ref, ea_ref, w_ref, wt_ref, wb_ref, m_ref):
    eb = jnp.dot(ea_ref[...], wb_ref[...], preferred_element_type=jnp.float32)
    zj = jnp.dot(xj_ref[...], wt_ref[...], preferred_element_type=jnp.float32) + eb
    oj = _sp(zj)
    w = w_ref[...]
    acc = w[:, 0:1] * oj[:, 0:64]
    acc += w[:, 1:2] * oj[:, 64:128]
    acc += w[:, 2:3] * oj[:, 128:192]
    acc += w[:, 3:4] * oj[:, 192:256]
    m_ref[...] = acc * 0.25


def _passC(xj, ea, w, wt, wb):
    return pl.pallas_call(
        _passC_body,
        grid=(_NT,),
        in_specs=[
            pl.BlockSpec((_TILE, 64), lambda t: (t, 0)),
            pl.BlockSpec((_TILE, 64), lambda t: (t, 0)),
            pl.BlockSpec((_TILE, 4), lambda t: (t, 0)),
            pl.BlockSpec((64, 256), lambda t: (0, 0)),
            pl.BlockSpec((64, 256), lambda t: (0, 0)),
        ],
        out_specs=pl.BlockSpec((_TILE, 64), lambda t: (t, 0)),
        out_shape=jax.ShapeDtypeStruct((_E, 64), jnp.float32),
        interpret=_INTERPRET,
    )(xj, ea, w, wt, wb)


def _passD_body(agg_ref, bias_ref, gb_ref, h_ref):
    h = agg_ref[...] + bias_ref[...]
    mu = jnp.mean(h, axis=0, keepdims=True)
    var = jnp.mean(h * h, axis=0, keepdims=True) - mu * mu
    inv = lax.rsqrt(var + 1e-5)
    h_ref[...] = gb_ref[0:1, :] * (h - mu) * inv + gb_ref[1:2, :]


def _passD(agg, bias, gb):
    return pl.pallas_call(
        _passD_body,
        in_specs=[
            pl.BlockSpec((_N, 64), lambda: (0, 0)),
            pl.BlockSpec((1, 64), lambda: (0, 0)),
            pl.BlockSpec((2, 64), lambda: (0, 0)),
        ],
        out_specs=pl.BlockSpec((_N, 64), lambda: (0, 0)),
        out_shape=jax.ShapeDtypeStruct((_N, 64), jnp.float32),
        interpret=_INTERPRET,
    )(agg, bias, gb)


def _final_body(h_ref, nb_ref, gf_ref, w1h_ref, w1g_ref, b1_ref, w2_ref,
                b2_ref, pw_ref, pb_ref, ow_ref, ob_ref, out_ref):
    h = h_ref[...]
    nb = nb_ref[...]
    onehot = (nb == lax.broadcasted_iota(jnp.int32, (1, _G), 1)).astype(jnp.float32)
    ge = jnp.dot(onehot, gf_ref[...], preferred_element_type=jnp.float32)
    a1 = _sp(jnp.dot(h, w1h_ref[...], preferred_element_type=jnp.float32)
             + jnp.dot(ge, w1g_ref[...], preferred_element_type=jnp.float32)
             + b1_ref[...])
    a = jnp.dot(a1, w2_ref[...], preferred_element_type=jnp.float32) + b2_ref[...]
    amax = jnp.max(a)
    e = jnp.exp(a - amax)
    sg = jnp.dot(onehot.T, e, preferred_element_type=jnp.float32)
    sn = jnp.dot(onehot, sg, preferred_element_type=jnp.float32)
    w = e / (sn + 1e-16)
    hw = h * w
    hg = jnp.dot(onehot.T, hw, preferred_element_type=jnp.float32)
    hg = _sp(jnp.dot(hg, pw_ref[...], preferred_element_type=jnp.float32) + pb_ref[...])
    out = jnp.dot(hg, ow_ref[...], preferred_element_type=jnp.float32) + ob_ref[...]
    out_ref[...] = out


def _final(h, nb2, gf, w1h, w1g, b1, w2, b2, pw, pb, ow, ob):
    return pl.pallas_call(
        _final_body,
        in_specs=[
            pl.BlockSpec((_N, 64), lambda: (0, 0)),
            pl.BlockSpec((_N, 1), lambda: (0, 0)),
            pl.BlockSpec((_G, 103), lambda: (0, 0)),
            pl.BlockSpec((64, 32), lambda: (0, 0)),
            pl.BlockSpec((103, 32), lambda: (0, 0)),
            pl.BlockSpec((1, 32), lambda: (0, 0)),
            pl.BlockSpec((32, 1), lambda: (0, 0)),
            pl.BlockSpec((1, 1), lambda: (0, 0)),
            pl.BlockSpec((64, 64), lambda: (0, 0)),
            pl.BlockSpec((1, 64), lambda: (0, 0)),
            pl.BlockSpec((64, 1), lambda: (0, 0)),
            pl.BlockSpec((1, 1), lambda: (0, 0)),
        ],
        out_specs=pl.BlockSpec((_G, 1), lambda: (0, 0)),
        out_shape=jax.ShapeDtypeStruct((_G, 1), jnp.float32),
        interpret=_INTERPRET,
    )(h, nb2, gf, w1h, w1g, b1, w2, b2, pw, pb, ow, ob)


def _stage0_x_body(x_ref, wx_ref, bx_ref, h_ref):
    h_ref[...] = jnp.dot(x_ref[...], wx_ref[...],
                         preferred_element_type=jnp.float32) + bx_ref[...]


def _stage0_x(x, wx, bx):
    return pl.pallas_call(
        _stage0_x_body,
        in_specs=[
            pl.BlockSpec((_N, 128), lambda: (0, 0)),
            pl.BlockSpec((128, 64), lambda: (0, 0)),
            pl.BlockSpec((1, 64), lambda: (0, 0)),
        ],
        out_specs=pl.BlockSpec((_N, 64), lambda: (0, 0)),
        out_shape=jax.ShapeDtypeStruct((_N, 64), jnp.float32),
        interpret=_INTERPRET,
    )(x, wx, bx)


def _stage0_e_body(ea_ref, we_ref, be_ref, out_ref):
    z = jnp.dot(ea_ref[...], we_ref[...],
                preferred_element_type=jnp.float32) + be_ref[...]
    out_ref[...] = jnp.where(z >= 0, z, 0.2 * z)


def _stage0_e(edge_attr, we, be):
    return pl.pallas_call(
        _stage0_e_body,
        grid=(_NT,),
        in_specs=[
            pl.BlockSpec((_TILE, 16), lambda t: (t, 0)),
            pl.BlockSpec((16, 64), lambda t: (0, 0)),
            pl.BlockSpec((1, 64), lambda t: (0, 0)),
        ],
        out_specs=pl.BlockSpec((_TILE, 64), lambda t: (t, 0)),
        out_shape=jax.ShapeDtypeStruct((_E, 64), jnp.float32),
        interpret=_INTERPRET,
    )(edge_attr, we, be)


def _nodeproj_body(h_ref, wt_ref, gi_ref):
    gi_ref[...] = jnp.dot(h_ref[...], wt_ref[...],
                          preferred_element_type=jnp.float32)


def _gather_rows(h, idx):
    return h[idx]


def _softmax_weights(spr, idx_i, consts):
    A = consts[0, :4]
    B = consts[1, :4]
    c = consts[2, :4]
    ev = (1.0 + jnp.exp(spr * A[None, :] + B[None, :])) * c[None, :]
    s = jax.ops.segment_sum(ev, idx_i, num_segments=_N)
    return ev / (s[idx_i] + 1e-16)


def _scatter_add(m, idx_i):
    return jax.ops.segment_sum(m, idx_i, num_segments=_N)


def kernel(x, edge_source, edge_target, edge_attr, global_fea, node_batch,
           W_x, b_x, W_e, b_e, conv_W, conv_att, conv_bias, bn1_g, bn1_b,
           bn_g, bn_b, ca_W1, ca_b1, ca_W2, ca_b2, post_W, post_b, out_W,
           out_b):
    idx_i = edge_source.astype(jnp.int32)
    idx_j = edge_target.astype(jnp.int32)

    h = _stage0_x(x, W_x, b_x[None, :])
    ea = _stage0_e(edge_attr, W_e, b_e[None, :])

    msel = (jnp.arange(256)[:, None] // 64 == jnp.arange(4)[None, :]).astype(jnp.float32)

    for l in range(3):
        wt = conv_W[l, :64, :]
        wb = conv_W[l, 64:, :]
        ai_flat = conv_att[l, :, :64].reshape(1, 256)
        aj_flat = conv_att[l, :, 64:].reshape(1, 256)
        gb1 = jnp.stack([
            jnp.pad(bn1_g[l], (0, 124)),
            jnp.pad(bn1_b[l], (0, 124)),
        ])

        xi = _gather_rows(h, idx_i)
        xj = _gather_rows(h, idx_j)
        spr, consts = _passA(xi, xj, ea, wt, wb, ai_flat, aj_flat, msel, gb1)
        w = _softmax_weights(spr, idx_i, consts)
        m = _passC(xj, ea, w, wt, wb)
        agg = _scatter_add(m, idx_i)
        gb = jnp.stack([bn_g[l], bn_b[l]])
        h = _passD(agg, conv_bias[l][None, :], gb)

    out = _final(h, node_batch.astype(jnp.int32)[:, None], global_fea,
                 ca_W1[:64, :], ca_W1[64:, :], ca_b1[None, :], ca_W2,
                 ca_b2[None, :], post_W, post_b[None, :], out_W, out_b[None, :])
    return out.reshape(-1)


# speedup vs baseline: 17.2546x; 1.8713x over previous
import functools

import jax
import jax.numpy as jnp
from jax import lax
from jax.experimental import pallas as pl
from jax.experimental.pallas import tpu as pltpu
from jax.experimental.pallas import tpu_sc as plsc

_N, _E, _G = 10000, 160000, 128
_TILE = 3200
_NT = _E // _TILE

_INTERPRET = False

_NC, _NS = 2, 16
_NW = _NC * _NS
_SC_MESH = plsc.VectorSubcoreMesh(core_axis_name="c", subcore_axis_name="s")


def _sp(x):
    return jnp.maximum(x, 0.0) + jnp.log(1.0 + jnp.exp(-jnp.abs(x)))


def _passA_body(xi_ref, xj_ref, ea_ref, wt_ref, wb_ref, ai_ref, aj_ref,
                msel_ref, gb_ref, spr_ref, consts_ref, acc_ref):
    t = pl.program_id(0)
    eb = jnp.dot(ea_ref[...], wb_ref[...], preferred_element_type=jnp.float32)
    zi = jnp.dot(xi_ref[...], wt_ref[...], preferred_element_type=jnp.float32) + eb
    zj = jnp.dot(xj_ref[...], wt_ref[...], preferred_element_type=jnp.float32) + eb
    oi = _sp(zi)
    oj = _sp(zj)
    prod = oi * ai_ref[...] + oj * aj_ref[...]
    a_raw = jnp.dot(prod, msel_ref[...], preferred_element_type=jnp.float32)
    spr = _sp(a_raw)
    spr_ref[...] = jnp.concatenate(
        [spr, jnp.zeros((spr.shape[0], 12), jnp.float32)], axis=1)

    pad = jnp.zeros((124,), jnp.float32)
    s1 = jnp.concatenate([jnp.sum(spr, 0), pad])[None, :]
    s2 = jnp.concatenate([jnp.sum(spr * spr, 0), pad])[None, :]
    mn = jnp.concatenate([jnp.min(spr, 0), jnp.full((124,), jnp.inf, jnp.float32)])[None, :]
    mx = jnp.concatenate([jnp.max(spr, 0), jnp.full((124,), -jnp.inf, jnp.float32)])[None, :]

    @pl.when(t == 0)
    def _():
        acc_ref[0:1, :] = s1
        acc_ref[1:2, :] = s2
        acc_ref[2:3, :] = mn
        acc_ref[3:4, :] = mx

    @pl.when(t > 0)
    def _():
        acc_ref[0:1, :] = acc_ref[0:1, :] + s1
        acc_ref[1:2, :] = acc_ref[1:2, :] + s2
        acc_ref[2:3, :] = jnp.minimum(acc_ref[2:3, :], mn)
        acc_ref[3:4, :] = jnp.maximum(acc_ref[3:4, :], mx)

    @pl.when(t == _NT - 1)
    def _():
        g = gb_ref[0:1, :]
        b = gb_ref[1:2, :]
        mu = acc_ref[0:1, :] / _E
        var = acc_ref[1:2, :] / _E - mu * mu
        inv = lax.rsqrt(var + 1e-5)
        A = g * inv
        B = b - g * mu * inv
        y_hi = jnp.maximum(A * acc_ref[3:4, :] + B, A * acc_ref[2:3, :] + B)
        c = 1.0 / (1.0 + jnp.exp(y_hi))
        consts_ref[0:1, :] = A
        consts_ref[1:2, :] = B
        consts_ref[2:3, :] = c


def _passA(xi, xj, ea, wt, wb, ai_flat, aj_flat, msel, gb):
    return pl.pallas_call(
        _passA_body,
        grid=(_NT,),
        in_specs=[
            pl.BlockSpec((_TILE, 64), lambda t: (t, 0)),
            pl.BlockSpec((_TILE, 64), lambda t: (t, 0)),
            pl.BlockSpec((_TILE, 64), lambda t: (t, 0)),
            pl.BlockSpec((64, 256), lambda t: (0, 0)),
            pl.BlockSpec((64, 256), lambda t: (0, 0)),
            pl.BlockSpec((1, 256), lambda t: (0, 0)),
            pl.BlockSpec((1, 256), lambda t: (0, 0)),
            pl.BlockSpec((256, 4), lambda t: (0, 0)),
            pl.BlockSpec((2, 128), lambda t: (0, 0)),
        ],
        out_specs=[
            pl.BlockSpec((_TILE, 16), lambda t: (t, 0)),
            pl.BlockSpec((4, 128), lambda t: (0, 0)),
        ],
        out_shape=[
            jax.ShapeDtypeStruct((_E, 16), jnp.float32),
            jax.ShapeDtypeStruct((4, 128), jnp.float32),
        ],
        scratch_shapes=[pltpu.VMEM((4, 128), jnp.float32)],
        interpret=_INTERPRET,
    )(xi, xj, ea, wt, wb, ai_flat, aj_flat, msel, gb)


def _passC_body(xj_ref, ea_ref, ev_ref, sv_ref, wt_ref, wb_ref, m_ref):
    eb = jnp.dot(ea_ref[...], wb_ref[...], preferred_element_type=jnp.float32)
    zj = jnp.dot(xj_ref[...], wt_ref[...], preferred_element_type=jnp.float32) + eb
    oj = _sp(zj)
    w = ev_ref[:, 0:4] / (sv_ref[:, 0:4] + 1e-16)
    acc = w[:, 0:1] * oj[:, 0:64]
    acc += w[:, 1:2] * oj[:, 64:128]
    acc += w[:, 2:3] * oj[:, 128:192]
    acc += w[:, 3:4] * oj[:, 192:256]
    m_ref[...] = acc * 0.25


def _passC(xj, ea, ev, sv, wt, wb):
    return pl.pallas_call(
        _passC_body,
        grid=(_NT,),
        in_specs=[
            pl.BlockSpec((_TILE, 64), lambda t: (t, 0)),
            pl.BlockSpec((_TILE, 64), lambda t: (t, 0)),
            pl.BlockSpec((_TILE, 16), lambda t: (t, 0)),
            pl.BlockSpec((_TILE, 16), lambda t: (t, 0)),
            pl.BlockSpec((64, 256), lambda t: (0, 0)),
            pl.BlockSpec((64, 256), lambda t: (0, 0)),
        ],
        out_specs=pl.BlockSpec((_TILE, 64), lambda t: (t, 0)),
        out_shape=jax.ShapeDtypeStruct((_E, 64), jnp.float32),
        interpret=_INTERPRET,
    )(xj, ea, ev, sv, wt, wb)


def _passD_body(agg_ref, bias_ref, gb_ref, h_ref):
    h = agg_ref[0] + agg_ref[1] + bias_ref[...]
    mu = jnp.mean(h, axis=0, keepdims=True)
    var = jnp.mean(h * h, axis=0, keepdims=True) - mu * mu
    inv = lax.rsqrt(var + 1e-5)
    h_ref[...] = gb_ref[0:1, :] * (h - mu) * inv + gb_ref[1:2, :]


def _passD(agg, bias, gb):
    return pl.pallas_call(
        _passD_body,
        in_specs=[
            pl.BlockSpec((2, _N, 64), lambda: (0, 0, 0)),
            pl.BlockSpec((1, 64), lambda: (0, 0)),
            pl.BlockSpec((2, 64), lambda: (0, 0)),
        ],
        out_specs=pl.BlockSpec((_N, 64), lambda: (0, 0)),
        out_shape=jax.ShapeDtypeStruct((_N, 64), jnp.float32),
        interpret=_INTERPRET,
    )(agg, bias, gb)


def _final_body(h_ref, nb_ref, gf_ref, w1h_ref, w1g_ref, b1_ref, w2_ref,
                b2_ref, pw_ref, pb_ref, ow_ref, ob_ref, out_ref):
    h = h_ref[...]
    nb = nb_ref[...]
    onehot = (nb == lax.broadcasted_iota(jnp.int32, (1, _G), 1)).astype(jnp.float32)
    ge = jnp.dot(onehot, gf_ref[...], preferred_element_type=jnp.float32)
    a1 = _sp(jnp.dot(h, w1h_ref[...], preferred_element_type=jnp.float32)
             + jnp.dot(ge, w1g_ref[...], preferred_element_type=jnp.float32)
             + b1_ref[...])
    a = jnp.dot(a1, w2_ref[...], preferred_element_type=jnp.float32) + b2_ref[...]
    amax = jnp.max(a)
    e = jnp.exp(a - amax)
    sg = jnp.dot(onehot.T, e, preferred_element_type=jnp.float32)
    sn = jnp.dot(onehot, sg, preferred_element_type=jnp.float32)
    w = e / (sn + 1e-16)
    hw = h * w
    hg = jnp.dot(onehot.T, hw, preferred_element_type=jnp.float32)
    hg = _sp(jnp.dot(hg, pw_ref[...], preferred_element_type=jnp.float32) + pb_ref[...])
    out = jnp.dot(hg, ow_ref[...], preferred_element_type=jnp.float32) + ob_ref[...]
    out_ref[...] = out


def _final(h, nb2, gf, w1h, w1g, b1, w2, b2, pw, pb, ow, ob):
    return pl.pallas_call(
        _final_body,
        in_specs=[
            pl.BlockSpec((_N, 64), lambda: (0, 0)),
            pl.BlockSpec((_N, 1), lambda: (0, 0)),
            pl.BlockSpec((_G, 103), lambda: (0, 0)),
            pl.BlockSpec((64, 32), lambda: (0, 0)),
            pl.BlockSpec((103, 32), lambda: (0, 0)),
            pl.BlockSpec((1, 32), lambda: (0, 0)),
            pl.BlockSpec((32, 1), lambda: (0, 0)),
            pl.BlockSpec((1, 1), lambda: (0, 0)),
            pl.BlockSpec((64, 64), lambda: (0, 0)),
            pl.BlockSpec((1, 64), lambda: (0, 0)),
            pl.BlockSpec((64, 1), lambda: (0, 0)),
            pl.BlockSpec((1, 1), lambda: (0, 0)),
        ],
        out_specs=pl.BlockSpec((_G, 1), lambda: (0, 0)),
        out_shape=jax.ShapeDtypeStruct((_G, 1), jnp.float32),
        interpret=_INTERPRET,
    )(h, nb2, gf, w1h, w1g, b1, w2, b2, pw, pb, ow, ob)


def _stage0_x_body(x_ref, wx_ref, bx_ref, h_ref):
    h_ref[...] = jnp.dot(x_ref[...], wx_ref[...],
                         preferred_element_type=jnp.float32) + bx_ref[...]


def _stage0_x(x, wx, bx):
    return pl.pallas_call(
        _stage0_x_body,
        in_specs=[
            pl.BlockSpec((_N, 128), lambda: (0, 0)),
            pl.BlockSpec((128, 64), lambda: (0, 0)),
            pl.BlockSpec((1, 64), lambda: (0, 0)),
        ],
        out_specs=pl.BlockSpec((_N, 64), lambda: (0, 0)),
        out_shape=jax.ShapeDtypeStruct((_N, 64), jnp.float32),
        interpret=_INTERPRET,
    )(x, wx, bx)


def _stage0_e_body(ea_ref, we_ref, be_ref, out_ref):
    z = jnp.dot(ea_ref[...], we_ref[...],
                preferred_element_type=jnp.float32) + be_ref[...]
    out_ref[...] = jnp.where(z >= 0, z, 0.2 * z)


def _stage0_e(edge_attr, we, be):
    return pl.pallas_call(
        _stage0_e_body,
        grid=(_NT,),
        in_specs=[
            pl.BlockSpec((_TILE, 16), lambda t: (t, 0)),
            pl.BlockSpec((16, 64), lambda t: (0, 0)),
            pl.BlockSpec((1, 64), lambda t: (0, 0)),
        ],
        out_specs=pl.BlockSpec((_TILE, 64), lambda t: (t, 0)),
        out_shape=jax.ShapeDtypeStruct((_E, 64), jnp.float32),
        interpret=_INTERPRET,
    )(edge_attr, we, be)


def _nodeproj_body(h_ref, wt_ref, gi_ref):
    gi_ref[...] = jnp.dot(h_ref[...], wt_ref[...],
                          preferred_element_type=jnp.float32)


def _passB_body(spr_ref, consts_ref, ev_ref):
    A = consts_ref[0:1, 0:16]
    B = consts_ref[1:2, 0:16]
    c = consts_ref[2:3, 0:16]
    ev = (1.0 + jnp.exp(spr_ref[...] * A + B)) * c
    mask = lax.broadcasted_iota(jnp.int32, (1, 16), 1) < 4
    ev_ref[...] = jnp.where(mask, ev, 0.0)


def _passB(spr, consts):
    return pl.pallas_call(
        _passB_body,
        grid=(_NT,),
        in_specs=[
            pl.BlockSpec((_TILE, 16), lambda t: (t, 0)),
            pl.BlockSpec((4, 128), lambda t: (0, 0)),
        ],
        out_specs=pl.BlockSpec((_TILE, 16), lambda t: (t, 0)),
        out_shape=jax.ShapeDtypeStruct((_E, 16), jnp.float32),
        interpret=_INTERPRET,
    )(spr, consts)


_CH = 128
_NCH = _E // _CH
_NPS = _N // _NS
_SC_CP = pltpu.CompilerParams(use_tc_tiling_on_sc=False)


def _sc_gather2(h, ii2, ij2):
    @functools.partial(
        pl.kernel,
        out_type=[jax.ShapeDtypeStruct((_E, 64), jnp.float32),
                  jax.ShapeDtypeStruct((_E, 64), jnp.float32)],
        mesh=_SC_MESH,
        compiler_params=_SC_CP,
        scratch_types=[pltpu.VMEM((_CH,), jnp.int32),
                       pltpu.VMEM((_CH, 64), jnp.float32),
                       pltpu.SemaphoreType.DMA],
    )
    def k(h_hbm, ii_hbm, ij_hbm, xi_hbm, xj_hbm, idx_v, rows_v, sem):
        wid = lax.axis_index("s") * _NC + lax.axis_index("c")

        def body(i, _):
            t = wid + _NW * i

            @pl.when(t < _NCH)
            def _():
                pltpu.sync_copy(ii_hbm.at[t], idx_v)
                pltpu.async_copy(h_hbm.at[idx_v], rows_v, sem).wait()
                pltpu.sync_copy(rows_v, xi_hbm.at[pl.ds(t * _CH, _CH)])
                pltpu.sync_copy(ij_hbm.at[t], idx_v)
                pltpu.async_copy(h_hbm.at[idx_v], rows_v, sem).wait()
                pltpu.sync_copy(rows_v, xj_hbm.at[pl.ds(t * _CH, _CH)])
            return 0

        lax.fori_loop(0, (_NCH + _NW - 1) // _NW, body, 0)

    return k(h, ii2, ij2)


def _sc_softmax_denom(ev16, ii2, zeros16):
    @functools.partial(
        pl.kernel,
        out_type=jax.ShapeDtypeStruct((_E, 16), jnp.float32),
        mesh=_SC_MESH,
        compiler_params=_SC_CP,
        scratch_types=[pltpu.VMEM((_CH,), jnp.int32),
                       pltpu.VMEM((_CH, 16), jnp.float32),
                       pltpu.VMEM_SHARED((_N, 16), jnp.float32),
                       pltpu.SemaphoreType.DMA],
    )
    def k(ev_hbm, ii_hbm, z_hbm, sv_hbm, idx_v, rows_v, acc_sh, sem):
        cid = lax.axis_index("c")
        sid = lax.axis_index("s")
        pltpu.sync_copy(z_hbm.at[pl.ds(sid * _NPS, _NPS)],
                        acc_sh.at[pl.ds(sid * _NPS, _NPS)])
        plsc.subcore_barrier()

        def body1(i, _):
            t = sid + _NS * i

            @pl.when(t < _NCH)
            def _():
                pltpu.sync_copy(ii_hbm.at[t], idx_v)
                pltpu.sync_copy(ev_hbm.at[pl.ds(t * _CH, _CH)], rows_v)
                pltpu.sync_copy(rows_v, acc_sh.at[idx_v], add=True)
            return 0

        lax.fori_loop(0, (_NCH + _NS - 1) // _NS, body1, 0)
        plsc.subcore_barrier()

        def body2(i, _):
            t = (cid * _NS + sid) + _NW * i

            @pl.when(t < _NCH)
            def _():
                pltpu.sync_copy(ii_hbm.at[t], idx_v)
                pltpu.async_copy(acc_sh.at[idx_v], rows_v, sem).wait()
                pltpu.sync_copy(rows_v, sv_hbm.at[pl.ds(t * _CH, _CH)])
            return 0

        lax.fori_loop(0, (_NCH + _NW - 1) // _NW, body2, 0)

    return k(ev16, ii2, zeros16)


def _sc_scatter_m(m, ii2, zeros64):
    @functools.partial(
        pl.kernel,
        out_type=jax.ShapeDtypeStruct((2, _N, 64), jnp.float32),
        mesh=_SC_MESH,
        compiler_params=_SC_CP,
        scratch_types=[pltpu.VMEM((_CH,), jnp.int32),
                       pltpu.VMEM((_CH, 64), jnp.float32),
                       pltpu.VMEM((_NPS, 64), jnp.float32),
                       pltpu.VMEM_SHARED((_N, 64), jnp.float32),
                       pltpu.SemaphoreType.DMA],
    )
    def k(m_hbm, ii_hbm, z_hbm, out_hbm, idx_v, rows_v, stripe_v, acc_sh, sem):
        cid = lax.axis_index("c")
        sid = lax.axis_index("s")
        pltpu.sync_copy(z_hbm.at[pl.ds(sid * _NPS, _NPS)],
                        acc_sh.at[pl.ds(sid * _NPS, _NPS)])
        plsc.subcore_barrier()

        def body(i, _):
            t = (cid * _NS + sid) + _NW * i

            @pl.when(t < _NCH)
            def _():
                pltpu.sync_copy(ii_hbm.at[t], idx_v)
                pltpu.sync_copy(m_hbm.at[pl.ds(t * _CH, _CH)], rows_v)
                pltpu.sync_copy(rows_v, acc_sh.at[idx_v], add=True)
            return 0

        lax.fori_loop(0, (_NCH + _NW - 1) // _NW, body, 0)
        plsc.subcore_barrier()
        pltpu.sync_copy(acc_sh.at[pl.ds(sid * _NPS, _NPS)], stripe_v)
        pltpu.sync_copy(stripe_v, out_hbm.at[cid, pl.ds(sid * _NPS, _NPS)])

    return k(m, ii2, zeros64)


def kernel(x, edge_source, edge_target, edge_attr, global_fea, node_batch,
           W_x, b_x, W_e, b_e, conv_W, conv_att, conv_bias, bn1_g, bn1_b,
           bn_g, bn_b, ca_W1, ca_b1, ca_W2, ca_b2, post_W, post_b, out_W,
           out_b):
    idx_i = edge_source.astype(jnp.int32)
    idx_j = edge_target.astype(jnp.int32)

    h = _stage0_x(x, W_x, b_x[None, :])
    ea = _stage0_e(edge_attr, W_e, b_e[None, :])

    msel = (jnp.arange(256)[:, None] // 64 == jnp.arange(4)[None, :]).astype(jnp.float32)
    ii2 = idx_i.reshape(_NCH, _CH)
    ij2 = idx_j.reshape(_NCH, _CH)
    zeros16 = jnp.zeros((_N, 16), jnp.float32)
    zeros64 = jnp.zeros((_N, 64), jnp.float32)

    for l in range(3):
        wt = conv_W[l, :64, :]
        wb = conv_W[l, 64:, :]
        ai_flat = conv_att[l, :, :64].reshape(1, 256)
        aj_flat = conv_att[l, :, 64:].reshape(1, 256)
        gb1 = jnp.stack([
            jnp.pad(bn1_g[l], (0, 124)),
            jnp.pad(bn1_b[l], (0, 124)),
        ])

        xi, xj = _sc_gather2(h, ii2, ij2)
        spr16, consts = _passA(xi, xj, ea, wt, wb, ai_flat, aj_flat, msel, gb1)
        ev16 = _passB(spr16, consts)
        sv16 = _sc_softmax_denom(ev16, ii2, zeros16)
        m = _passC(xj, ea, ev16, sv16, wt, wb)
        agg = _sc_scatter_m(m, ii2, zeros64)
        gb = jnp.stack([bn_g[l], bn_b[l]])
        h = _passD(agg, conv_bias[l][None, :], gb)

    out = _final(h, node_batch.astype(jnp.int32)[:, None], global_fea,
                 ca_W1[:64, :], ca_W1[64:, :], ca_b1[None, :], ca_W2,
                 ca_b2[None, :], post_W, post_b[None, :], out_W, out_b[None, :])
    return out.reshape(-1)


# speedup vs baseline: 21.3071x; 1.2349x over previous
import functools

import jax
import jax.numpy as jnp
from jax import lax
from jax.experimental import pallas as pl
from jax.experimental.pallas import tpu as pltpu
from jax.experimental.pallas import tpu_sc as plsc

_N, _E, _G = 10000, 160000, 128
_TILE = 3200
_NT = _E // _TILE

_INTERPRET = False

_NC, _NS = 2, 16
_NW = _NC * _NS
_SC_MESH = plsc.VectorSubcoreMesh(core_axis_name="c", subcore_axis_name="s")


def _sp(x):
    return jnp.maximum(x, 0.0) + jnp.log(1.0 + jnp.exp(-jnp.abs(x)))


def _passA_body(xi_ref, xj_ref, ea_ref, wt_ref, wb_ref, ai_ref, aj_ref,
                msel_ref, gb_ref, spr_ref, consts_ref, acc_ref):
    t = pl.program_id(0)
    eb = jnp.dot(ea_ref[...], wb_ref[...], preferred_element_type=jnp.float32)
    zi = jnp.dot(xi_ref[...], wt_ref[...], preferred_element_type=jnp.float32) + eb
    zj = jnp.dot(xj_ref[...], wt_ref[...], preferred_element_type=jnp.float32) + eb
    oi = _sp(zi)
    oj = _sp(zj)
    prod = oi * ai_ref[...] + oj * aj_ref[...]
    a_raw = jnp.dot(prod, msel_ref[...], preferred_element_type=jnp.float32)
    spr = _sp(a_raw)
    spr_ref[...] = jnp.concatenate(
        [spr, jnp.zeros((spr.shape[0], 12), jnp.float32)], axis=1)

    pad = jnp.zeros((124,), jnp.float32)
    s1 = jnp.concatenate([jnp.sum(spr, 0), pad])[None, :]
    s2 = jnp.concatenate([jnp.sum(spr * spr, 0), pad])[None, :]
    mn = jnp.concatenate([jnp.min(spr, 0), jnp.full((124,), jnp.inf, jnp.float32)])[None, :]
    mx = jnp.concatenate([jnp.max(spr, 0), jnp.full((124,), -jnp.inf, jnp.float32)])[None, :]

    @pl.when(t == 0)
    def _():
        acc_ref[0:1, :] = s1
        acc_ref[1:2, :] = s2
        acc_ref[2:3, :] = mn
        acc_ref[3:4, :] = mx

    @pl.when(t > 0)
    def _():
        acc_ref[0:1, :] = acc_ref[0:1, :] + s1
        acc_ref[1:2, :] = acc_ref[1:2, :] + s2
        acc_ref[2:3, :] = jnp.minimum(acc_ref[2:3, :], mn)
        acc_ref[3:4, :] = jnp.maximum(acc_ref[3:4, :], mx)

    @pl.when(t == _NT - 1)
    def _():
        g = gb_ref[0:1, :]
        b = gb_ref[1:2, :]
        mu = acc_ref[0:1, :] / _E
        var = acc_ref[1:2, :] / _E - mu * mu
        inv = lax.rsqrt(var + 1e-5)
        A = g * inv
        B = b - g * mu * inv
        y_hi = jnp.maximum(A * acc_ref[3:4, :] + B, A * acc_ref[2:3, :] + B)
        c = 1.0 / (1.0 + jnp.exp(y_hi))
        consts_ref[0:1, :] = A
        consts_ref[1:2, :] = B
        consts_ref[2:3, :] = c


def _passA(xi, xj, ea, wt, wb, ai_flat, aj_flat, msel, gb):
    return pl.pallas_call(
        _passA_body,
        grid=(_NT,),
        in_specs=[
            pl.BlockSpec((_TILE, 64), lambda t: (t, 0)),
            pl.BlockSpec((_TILE, 64), lambda t: (t, 0)),
            pl.BlockSpec((_TILE, 64), lambda t: (t, 0)),
            pl.BlockSpec((64, 256), lambda t: (0, 0)),
            pl.BlockSpec((64, 256), lambda t: (0, 0)),
            pl.BlockSpec((1, 256), lambda t: (0, 0)),
            pl.BlockSpec((1, 256), lambda t: (0, 0)),
            pl.BlockSpec((256, 4), lambda t: (0, 0)),
            pl.BlockSpec((2, 128), lambda t: (0, 0)),
        ],
        out_specs=[
            pl.BlockSpec((_TILE, 16), lambda t: (t, 0)),
            pl.BlockSpec((4, 128), lambda t: (0, 0)),
        ],
        out_shape=[
            jax.ShapeDtypeStruct((_E, 16), jnp.float32),
            jax.ShapeDtypeStruct((4, 128), jnp.float32),
        ],
        scratch_shapes=[pltpu.VMEM((4, 128), jnp.float32)],
        interpret=_INTERPRET,
    )(xi, xj, ea, wt, wb, ai_flat, aj_flat, msel, gb)


def _passC_body(xj_ref, ea_ref, ev_ref, sv_ref, wt_ref, wb_ref, m_ref):
    eb = jnp.dot(ea_ref[...], wb_ref[...], preferred_element_type=jnp.float32)
    zj = jnp.dot(xj_ref[...], wt_ref[...], preferred_element_type=jnp.float32) + eb
    oj = _sp(zj)
    w = ev_ref[:, 0:4] / (sv_ref[:, 0:4] + 1e-16)
    acc = w[:, 0:1] * oj[:, 0:64]
    acc += w[:, 1:2] * oj[:, 64:128]
    acc += w[:, 2:3] * oj[:, 128:192]
    acc += w[:, 3:4] * oj[:, 192:256]
    m_ref[...] = acc * 0.25


def _passC(xj, ea, ev, sv, wt, wb):
    return pl.pallas_call(
        _passC_body,
        grid=(_NT,),
        in_specs=[
            pl.BlockSpec((_TILE, 64), lambda t: (t, 0)),
            pl.BlockSpec((_TILE, 64), lambda t: (t, 0)),
            pl.BlockSpec((_TILE, 16), lambda t: (t, 0)),
            pl.BlockSpec((_TILE, 16), lambda t: (t, 0)),
            pl.BlockSpec((64, 256), lambda t: (0, 0)),
            pl.BlockSpec((64, 256), lambda t: (0, 0)),
        ],
        out_specs=pl.BlockSpec((_TILE, 64), lambda t: (t, 0)),
        out_shape=jax.ShapeDtypeStruct((_E, 64), jnp.float32),
        interpret=_INTERPRET,
    )(xj, ea, ev, sv, wt, wb)


def _passD_body(agg_ref, bias_ref, gb_ref, h_ref):
    h = agg_ref[0] + agg_ref[1] + bias_ref[...]
    mu = jnp.mean(h, axis=0, keepdims=True)
    var = jnp.mean(h * h, axis=0, keepdims=True) - mu * mu
    inv = lax.rsqrt(var + 1e-5)
    h_ref[...] = gb_ref[0:1, :] * (h - mu) * inv + gb_ref[1:2, :]


def _passD(agg, bias, gb):
    return pl.pallas_call(
        _passD_body,
        in_specs=[
            pl.BlockSpec((2, _N, 64), lambda: (0, 0, 0)),
            pl.BlockSpec((1, 64), lambda: (0, 0)),
            pl.BlockSpec((2, 64), lambda: (0, 0)),
        ],
        out_specs=pl.BlockSpec((_N, 64), lambda: (0, 0)),
        out_shape=jax.ShapeDtypeStruct((_N, 64), jnp.float32),
        interpret=_INTERPRET,
    )(agg, bias, gb)


def _final_body(h_ref, nb_ref, gf_ref, w1h_ref, w1g_ref, b1_ref, w2_ref,
                b2_ref, pw_ref, pb_ref, ow_ref, ob_ref, out_ref):
    h = h_ref[...]
    nb = nb_ref[...]
    onehot = (nb == lax.broadcasted_iota(jnp.int32, (1, _G), 1)).astype(jnp.float32)
    ge = jnp.dot(onehot, gf_ref[...], preferred_element_type=jnp.float32)
    a1 = _sp(jnp.dot(h, w1h_ref[...], preferred_element_type=jnp.float32)
             + jnp.dot(ge, w1g_ref[...], preferred_element_type=jnp.float32)
             + b1_ref[...])
    a = jnp.dot(a1, w2_ref[...], preferred_element_type=jnp.float32) + b2_ref[...]
    amax = jnp.max(a)
    e = jnp.exp(a - amax)
    sg = jnp.dot(onehot.T, e, preferred_element_type=jnp.float32)
    sn = jnp.dot(onehot, sg, preferred_element_type=jnp.float32)
    w = e / (sn + 1e-16)
    hw = h * w
    hg = jnp.dot(onehot.T, hw, preferred_element_type=jnp.float32)
    hg = _sp(jnp.dot(hg, pw_ref[...], preferred_element_type=jnp.float32) + pb_ref[...])
    out = jnp.dot(hg, ow_ref[...], preferred_element_type=jnp.float32) + ob_ref[...]
    out_ref[...] = out


def _final(h, nb2, gf, w1h, w1g, b1, w2, b2, pw, pb, ow, ob):
    return pl.pallas_call(
        _final_body,
        in_specs=[
            pl.BlockSpec((_N, 64), lambda: (0, 0)),
            pl.BlockSpec((_N, 1), lambda: (0, 0)),
            pl.BlockSpec((_G, 103), lambda: (0, 0)),
            pl.BlockSpec((64, 32), lambda: (0, 0)),
            pl.BlockSpec((103, 32), lambda: (0, 0)),
            pl.BlockSpec((1, 32), lambda: (0, 0)),
            pl.BlockSpec((32, 1), lambda: (0, 0)),
            pl.BlockSpec((1, 1), lambda: (0, 0)),
            pl.BlockSpec((64, 64), lambda: (0, 0)),
            pl.BlockSpec((1, 64), lambda: (0, 0)),
            pl.BlockSpec((64, 1), lambda: (0, 0)),
            pl.BlockSpec((1, 1), lambda: (0, 0)),
        ],
        out_specs=pl.BlockSpec((_G, 1), lambda: (0, 0)),
        out_shape=jax.ShapeDtypeStruct((_G, 1), jnp.float32),
        interpret=_INTERPRET,
    )(h, nb2, gf, w1h, w1g, b1, w2, b2, pw, pb, ow, ob)


def _stage0_x_body(x_ref, wx_ref, bx_ref, h_ref):
    h_ref[...] = jnp.dot(x_ref[...], wx_ref[...],
                         preferred_element_type=jnp.float32) + bx_ref[...]


def _stage0_x(x, wx, bx):
    return pl.pallas_call(
        _stage0_x_body,
        in_specs=[
            pl.BlockSpec((_N, 128), lambda: (0, 0)),
            pl.BlockSpec((128, 64), lambda: (0, 0)),
            pl.BlockSpec((1, 64), lambda: (0, 0)),
        ],
        out_specs=pl.BlockSpec((_N, 64), lambda: (0, 0)),
        out_shape=jax.ShapeDtypeStruct((_N, 64), jnp.float32),
        interpret=_INTERPRET,
    )(x, wx, bx)


def _stage0_e_body(ea_ref, we_ref, be_ref, out_ref):
    z = jnp.dot(ea_ref[...], we_ref[...],
                preferred_element_type=jnp.float32) + be_ref[...]
    out_ref[...] = jnp.where(z >= 0, z, 0.2 * z)


def _stage0_e(edge_attr, we, be):
    return pl.pallas_call(
        _stage0_e_body,
        grid=(_NT,),
        in_specs=[
            pl.BlockSpec((_TILE, 16), lambda t: (t, 0)),
            pl.BlockSpec((16, 64), lambda t: (0, 0)),
            pl.BlockSpec((1, 64), lambda t: (0, 0)),
        ],
        out_specs=pl.BlockSpec((_TILE, 64), lambda t: (t, 0)),
        out_shape=jax.ShapeDtypeStruct((_E, 64), jnp.float32),
        interpret=_INTERPRET,
    )(edge_attr, we, be)


def _nodeproj_body(h_ref, wt_ref, gi_ref):
    gi_ref[...] = jnp.dot(h_ref[...], wt_ref[...],
                          preferred_element_type=jnp.float32)


def _passB_body(spr_ref, consts_ref, ev_ref):
    A = consts_ref[0:1, 0:16]
    B = consts_ref[1:2, 0:16]
    c = consts_ref[2:3, 0:16]
    ev = (1.0 + jnp.exp(spr_ref[...] * A + B)) * c
    mask = lax.broadcasted_iota(jnp.int32, (1, 16), 1) < 4
    ev_ref[...] = jnp.where(mask, ev, 0.0)


def _passB(spr, consts):
    return pl.pallas_call(
        _passB_body,
        grid=(_NT,),
        in_specs=[
            pl.BlockSpec((_TILE, 16), lambda t: (t, 0)),
            pl.BlockSpec((4, 128), lambda t: (0, 0)),
        ],
        out_specs=pl.BlockSpec((_TILE, 16), lambda t: (t, 0)),
        out_shape=jax.ShapeDtypeStruct((_E, 16), jnp.float32),
        interpret=_INTERPRET,
    )(spr, consts)


_CH = 100
_NCH = _E // _CH
_CPW = _NCH // _NW
_CPT = _NCH // _NS
_NPS = _N // _NS
_NBUF = 5
_SC_CP = pltpu.CompilerParams(use_tc_tiling_on_sc=False)


def _sc_gather2(h, ii2, ij2):
    @functools.partial(
        pl.kernel,
        out_type=[jax.ShapeDtypeStruct((_E, 64), jnp.float32),
                  jax.ShapeDtypeStruct((_E, 64), jnp.float32)],
        mesh=_SC_MESH,
        compiler_params=_SC_CP,
        scratch_types=[pltpu.VMEM((_CPW, _CH), jnp.int32),
                       pltpu.VMEM((_NBUF, _CH, 64), jnp.float32),
                       pltpu.SemaphoreType.DMA((_NBUF,)),
                       pltpu.SemaphoreType.DMA((_NBUF,))],
    )
    def k(h_hbm, ii_hbm, ij_hbm, xi_hbm, xj_hbm, idx_v, rows_v, gsem, osem):
        wid = lax.axis_index("s") * _NC + lax.axis_index("c")
        cbase = wid * _CPW

        def run(src_hbm, dst_hbm):
            pltpu.sync_copy(src_hbm.at[pl.ds(cbase, _CPW)], idx_v)

            def group(g, _):
                for b in range(_NBUF):
                    i = g * _NBUF + b
                    t = cbase + i

                    @pl.when(g > 0)
                    def _():
                        pltpu.make_async_copy(
                            rows_v.at[b],
                            dst_hbm.at[pl.ds((t - _NBUF) * _CH, _CH)],
                            osem.at[b]).wait()
                    pltpu.async_copy(h_hbm.at[idx_v.at[i]], rows_v.at[b], gsem.at[b])
                for b in range(_NBUF):
                    i = g * _NBUF + b
                    t = cbase + i
                    pltpu.make_async_copy(h_hbm.at[idx_v.at[i]],
                                          rows_v.at[b], gsem.at[b]).wait()
                    pltpu.async_copy(rows_v.at[b],
                                     dst_hbm.at[pl.ds(t * _CH, _CH)], osem.at[b])
                return 0

            ng = _CPW // _NBUF
            lax.fori_loop(0, ng, group, 0)
            for b in range(_NBUF):
                t = cbase + (ng - 1) * _NBUF + b
                pltpu.make_async_copy(rows_v.at[b],
                                      dst_hbm.at[pl.ds(t * _CH, _CH)],
                                      osem.at[b]).wait()

        run(ii_hbm, xi_hbm)
        run(ij_hbm, xj_hbm)

    return k(h, ii2, ij2)


def _sc_softmax_denom(ev16, ii2, zeros16):
    @functools.partial(
        pl.kernel,
        out_type=jax.ShapeDtypeStruct((_E, 16), jnp.float32),
        mesh=_SC_MESH,
        compiler_params=_SC_CP,
        scratch_types=[pltpu.VMEM((_CPT, _CH), jnp.int32),
                       pltpu.VMEM((_NBUF, _CH, 16), jnp.float32),
                       pltpu.VMEM_SHARED((_N, 16), jnp.float32),
                       pltpu.SemaphoreType.DMA((_NBUF,)),
                       pltpu.SemaphoreType.DMA((_NBUF,))],
    )
    def k(ev_hbm, ii_hbm, z_hbm, sv_hbm, idx_v, rows_v, acc_sh, lsem, ssem):
        cid = lax.axis_index("c")
        sid = lax.axis_index("s")
        pltpu.sync_copy(z_hbm.at[pl.ds(sid * _NPS, _NPS)],
                        acc_sh.at[pl.ds(sid * _NPS, _NPS)])
        cbase1 = sid * _CPT
        pltpu.sync_copy(ii_hbm.at[pl.ds(cbase1, _CPT)], idx_v)
        plsc.subcore_barrier()

        def group1(g, _):
            for b in range(_NBUF):
                i = g * _NBUF + b
                t = cbase1 + i

                @pl.when(g > 0)
                def _():
                    pltpu.make_async_copy(rows_v.at[b],
                                          acc_sh.at[idx_v.at[i - _NBUF]],
                                          ssem.at[b]).wait()
                pltpu.async_copy(ev_hbm.at[pl.ds(t * _CH, _CH)],
                                 rows_v.at[b], lsem.at[b])
            for b in range(_NBUF):
                i = g * _NBUF + b
                t = cbase1 + i
                pltpu.make_async_copy(ev_hbm.at[pl.ds(t * _CH, _CH)],
                                      rows_v.at[b], lsem.at[b]).wait()
                pltpu.async_copy(rows_v.at[b], acc_sh.at[idx_v.at[i]],
                                 ssem.at[b], add=True)
            return 0

        ng1 = _CPT // _NBUF
        lax.fori_loop(0, ng1, group1, 0)
        for b in range(_NBUF):
            i = (ng1 - 1) * _NBUF + b
            pltpu.make_async_copy(rows_v.at[b], acc_sh.at[idx_v.at[i]],
                                  ssem.at[b]).wait()
        plsc.subcore_barrier()

        wid = cid * _NS + sid
        cbase2 = wid * _CPW
        pltpu.sync_copy(ii_hbm.at[pl.ds(cbase2, _CPW)], idx_v.at[pl.ds(0, _CPW)])

        def group2(g, _):
            for b in range(_NBUF):
                i = g * _NBUF + b
                t = cbase2 + i

                @pl.when(g > 0)
                def _():
                    pltpu.make_async_copy(
                        rows_v.at[b],
                        sv_hbm.at[pl.ds((t - _NBUF) * _CH, _CH)], ssem.at[b]).wait()
                pltpu.async_copy(acc_sh.at[idx_v.at[i]], rows_v.at[b], lsem.at[b])
            for b in range(_NBUF):
                i = g * _NBUF + b
                t = cbase2 + i
                pltpu.make_async_copy(acc_sh.at[idx_v.at[i]],
                                      rows_v.at[b], lsem.at[b]).wait()
                pltpu.async_copy(rows_v.at[b],
                                 sv_hbm.at[pl.ds(t * _CH, _CH)], ssem.at[b])
            return 0

        ng2 = _CPW // _NBUF
        lax.fori_loop(0, ng2, group2, 0)
        for b in range(_NBUF):
            t = cbase2 + (ng2 - 1) * _NBUF + b
            pltpu.make_async_copy(rows_v.at[b],
                                  sv_hbm.at[pl.ds(t * _CH, _CH)], ssem.at[b]).wait()

    return k(ev16, ii2, zeros16)


def _sc_scatter_m(m, ii2, zeros64):
    @functools.partial(
        pl.kernel,
        out_type=jax.ShapeDtypeStruct((2, _N, 64), jnp.float32),
        mesh=_SC_MESH,
        compiler_params=_SC_CP,
        scratch_types=[pltpu.VMEM((_CPW, _CH), jnp.int32),
                       pltpu.VMEM((_NBUF, _CH, 64), jnp.float32),
                       pltpu.VMEM((_NPS, 64), jnp.float32),
                       pltpu.VMEM_SHARED((_N, 64), jnp.float32),
                       pltpu.SemaphoreType.DMA((_NBUF,)),
                       pltpu.SemaphoreType.DMA((_NBUF,))],
    )
    def k(m_hbm, ii_hbm, z_hbm, out_hbm, idx_v, rows_v, stripe_v, acc_sh,
          lsem, ssem):
        cid = lax.axis_index("c")
        sid = lax.axis_index("s")
        wid = cid * _NS + sid
        cbase = wid * _CPW
        pltpu.sync_copy(z_hbm.at[pl.ds(sid * _NPS, _NPS)],
                        acc_sh.at[pl.ds(sid * _NPS, _NPS)])
        pltpu.sync_copy(ii_hbm.at[pl.ds(cbase, _CPW)], idx_v)
        plsc.subcore_barrier()

        def group(g, _):
            for b in range(_NBUF):
                i = g * _NBUF + b
                t = cbase + i

                @pl.when(g > 0)
                def _():
                    pltpu.make_async_copy(rows_v.at[b],
                                          acc_sh.at[idx_v.at[i - _NBUF]],
                                          ssem.at[b]).wait()
                pltpu.async_copy(m_hbm.at[pl.ds(t * _CH, _CH)],
                                 rows_v.at[b], lsem.at[b])
            for b in range(_NBUF):
                i = g * _NBUF + b
                t = cbase + i
                pltpu.make_async_copy(m_hbm.at[pl.ds(t * _CH, _CH)],
                                      rows_v.at[b], lsem.at[b]).wait()
                pltpu.async_copy(rows_v.at[b], acc_sh.at[idx_v.at[i]],
                                 ssem.at[b], add=True)
            return 0

        ng = _CPW // _NBUF
        lax.fori_loop(0, ng, group, 0)
        for b in range(_NBUF):
            i = (ng - 1) * _NBUF + b
            pltpu.make_async_copy(rows_v.at[b], acc_sh.at[idx_v.at[i]],
                                  ssem.at[b]).wait()
        plsc.subcore_barrier()
        pltpu.sync_copy(acc_sh.at[pl.ds(sid * _NPS, _NPS)], stripe_v)
        pltpu.sync_copy(stripe_v, out_hbm.at[cid, pl.ds(sid * _NPS, _NPS)])

    return k(m, ii2, zeros64)


def kernel(x, edge_source, edge_target, edge_attr, global_fea, node_batch,
           W_x, b_x, W_e, b_e, conv_W, conv_att, conv_bias, bn1_g, bn1_b,
           bn_g, bn_b, ca_W1, ca_b1, ca_W2, ca_b2, post_W, post_b, out_W,
           out_b):
    idx_i = edge_source.astype(jnp.int32)
    idx_j = edge_target.astype(jnp.int32)

    h = _stage0_x(x, W_x, b_x[None, :])
    ea = _stage0_e(edge_attr, W_e, b_e[None, :])

    msel = (jnp.arange(256)[:, None] // 64 == jnp.arange(4)[None, :]).astype(jnp.float32)
    ii2 = idx_i.reshape(_NCH, _CH)
    ij2 = idx_j.reshape(_NCH, _CH)
    zeros16 = jnp.zeros((_N, 16), jnp.float32)
    zeros64 = jnp.zeros((_N, 64), jnp.float32)

    for l in range(3):
        wt = conv_W[l, :64, :]
        wb = conv_W[l, 64:, :]
        ai_flat = conv_att[l, :, :64].reshape(1, 256)
        aj_flat = conv_att[l, :, 64:].reshape(1, 256)
        gb1 = jnp.stack([
            jnp.pad(bn1_g[l], (0, 124)),
            jnp.pad(bn1_b[l], (0, 124)),
        ])

        xi, xj = _sc_gather2(h, ii2, ij2)
        spr16, consts = _passA(xi, xj, ea, wt, wb, ai_flat, aj_flat, msel, gb1)
        ev16 = _passB(spr16, consts)
        sv16 = _sc_softmax_denom(ev16, ii2, zeros16)
        m = _passC(xj, ea, ev16, sv16, wt, wb)
        agg = _sc_scatter_m(m, ii2, zeros64)
        gb = jnp.stack([bn_g[l], bn_b[l]])
        h = _passD(agg, conv_bias[l][None, :], gb)

    out = _final(h, node_batch.astype(jnp.int32)[:, None], global_fea,
                 ca_W1[:64, :], ca_W1[64:, :], ca_b1[None, :], ca_W2,
                 ca_b2[None, :], post_W, post_b[None, :], out_W, out_b[None, :])
    return out.reshape(-1)


# speedup vs baseline: 21.4019x; 1.0044x over previous
import functools

import jax
import jax.numpy as jnp
from jax import lax
from jax.experimental import pallas as pl
from jax.experimental.pallas import tpu as pltpu
from jax.experimental.pallas import tpu_sc as plsc

_N, _E, _G = 10000, 160000, 128
_TILE = 3200
_NT = _E // _TILE

_INTERPRET = False

_NC, _NS = 2, 16
_NW = _NC * _NS
_SC_MESH = plsc.VectorSubcoreMesh(core_axis_name="c", subcore_axis_name="s")


def _sp(x):
    return jnp.maximum(x, 0.0) + jnp.log(1.0 + jnp.exp(-jnp.abs(x)))


def _passA_body(xi_ref, xj_ref, ea_ref, wbig_ref, ai_ref, aj_ref,
                msel_ref, gb_ref, spr_ref, consts_ref, acc_ref):
    t = pl.program_id(0)
    xje = jnp.concatenate([xi_ref[...], xj_ref[...], ea_ref[...]], axis=1)
    z = jnp.dot(xje, wbig_ref[...], preferred_element_type=jnp.float32)
    oi = _sp(z[:, 0:256])
    oj = _sp(z[:, 256:512])
    prod = oi * ai_ref[...] + oj * aj_ref[...]
    a_raw = jnp.dot(prod, msel_ref[...], preferred_element_type=jnp.float32)
    spr = _sp(a_raw)
    spr_ref[...] = jnp.concatenate(
        [spr, jnp.zeros((spr.shape[0], 12), jnp.float32)], axis=1)

    pad = jnp.zeros((124,), jnp.float32)
    s1 = jnp.concatenate([jnp.sum(spr, 0), pad])[None, :]
    s2 = jnp.concatenate([jnp.sum(spr * spr, 0), pad])[None, :]
    mn = jnp.concatenate([jnp.min(spr, 0), jnp.full((124,), jnp.inf, jnp.float32)])[None, :]
    mx = jnp.concatenate([jnp.max(spr, 0), jnp.full((124,), -jnp.inf, jnp.float32)])[None, :]

    @pl.when(t == 0)
    def _():
        acc_ref[0:1, :] = s1
        acc_ref[1:2, :] = s2
        acc_ref[2:3, :] = mn
        acc_ref[3:4, :] = mx

    @pl.when(t > 0)
    def _():
        acc_ref[0:1, :] = acc_ref[0:1, :] + s1
        acc_ref[1:2, :] = acc_ref[1:2, :] + s2
        acc_ref[2:3, :] = jnp.minimum(acc_ref[2:3, :], mn)
        acc_ref[3:4, :] = jnp.maximum(acc_ref[3:4, :], mx)

    @pl.when(t == _NT - 1)
    def _():
        g = gb_ref[0:1, :]
        b = gb_ref[1:2, :]
        mu = acc_ref[0:1, :] / _E
        var = acc_ref[1:2, :] / _E - mu * mu
        inv = lax.rsqrt(var + 1e-5)
        A = g * inv
        B = b - g * mu * inv
        y_hi = jnp.maximum(A * acc_ref[3:4, :] + B, A * acc_ref[2:3, :] + B)
        c = 1.0 / (1.0 + jnp.exp(y_hi))
        consts_ref[0:1, :] = A
        consts_ref[1:2, :] = B
        consts_ref[2:3, :] = c


def _passA(xi, xj, ea, wbig, ai_flat, aj_flat, msel, gb):
    return pl.pallas_call(
        _passA_body,
        grid=(_NT,),
        in_specs=[
            pl.BlockSpec((_TILE, 64), lambda t: (t, 0)),
            pl.BlockSpec((_TILE, 64), lambda t: (t, 0)),
            pl.BlockSpec((_TILE, 64), lambda t: (t, 0)),
            pl.BlockSpec((192, 512), lambda t: (0, 0)),
            pl.BlockSpec((1, 256), lambda t: (0, 0)),
            pl.BlockSpec((1, 256), lambda t: (0, 0)),
            pl.BlockSpec((256, 4), lambda t: (0, 0)),
            pl.BlockSpec((2, 128), lambda t: (0, 0)),
        ],
        out_specs=[
            pl.BlockSpec((_TILE, 16), lambda t: (t, 0)),
            pl.BlockSpec((4, 128), lambda t: (0, 0)),
        ],
        out_shape=[
            jax.ShapeDtypeStruct((_E, 16), jnp.float32),
            jax.ShapeDtypeStruct((4, 128), jnp.float32),
        ],
        scratch_shapes=[pltpu.VMEM((4, 128), jnp.float32)],
        interpret=_INTERPRET,
    )(xi, xj, ea, wbig, ai_flat, aj_flat, msel, gb)


def _passC_body(xj_ref, ea_ref, ev_ref, sv_ref, w2_ref, m_ref):
    xe = jnp.concatenate([xj_ref[...], ea_ref[...]], axis=1)
    zj = jnp.dot(xe, w2_ref[...], preferred_element_type=jnp.float32)
    oj = _sp(zj)
    w = ev_ref[:, 0:4] / (sv_ref[:, 0:4] + 1e-16)
    acc = w[:, 0:1] * oj[:, 0:64]
    acc += w[:, 1:2] * oj[:, 64:128]
    acc += w[:, 2:3] * oj[:, 128:192]
    acc += w[:, 3:4] * oj[:, 192:256]
    m_ref[...] = acc * 0.25


def _passC(xj, ea, ev, sv, w2):
    return pl.pallas_call(
        _passC_body,
        grid=(_NT,),
        in_specs=[
            pl.BlockSpec((_TILE, 64), lambda t: (t, 0)),
            pl.BlockSpec((_TILE, 64), lambda t: (t, 0)),
            pl.BlockSpec((_TILE, 16), lambda t: (t, 0)),
            pl.BlockSpec((_TILE, 16), lambda t: (t, 0)),
            pl.BlockSpec((128, 256), lambda t: (0, 0)),
        ],
        out_specs=pl.BlockSpec((_TILE, 64), lambda t: (t, 0)),
        out_shape=jax.ShapeDtypeStruct((_E, 64), jnp.float32),
        interpret=_INTERPRET,
    )(xj, ea, ev, sv, w2)


def _passD_body(agg_ref, bias_ref, gb_ref, h_ref):
    h = agg_ref[0] + agg_ref[1] + bias_ref[...]
    mu = jnp.mean(h, axis=0, keepdims=True)
    var = jnp.mean(h * h, axis=0, keepdims=True) - mu * mu
    inv = lax.rsqrt(var + 1e-5)
    h_ref[...] = gb_ref[0:1, :] * (h - mu) * inv + gb_ref[1:2, :]


def _passD(agg, bias, gb):
    return pl.pallas_call(
        _passD_body,
        in_specs=[
            pl.BlockSpec((2, _N, 64), lambda: (0, 0, 0)),
            pl.BlockSpec((1, 64), lambda: (0, 0)),
            pl.BlockSpec((2, 64), lambda: (0, 0)),
        ],
        out_specs=pl.BlockSpec((_N, 64), lambda: (0, 0)),
        out_shape=jax.ShapeDtypeStruct((_N, 64), jnp.float32),
        interpret=_INTERPRET,
    )(agg, bias, gb)


def _final_body(h_ref, nb_ref, gf_ref, w1h_ref, w1g_ref, b1_ref, w2_ref,
                b2_ref, pw_ref, pb_ref, ow_ref, ob_ref, out_ref):
    h = h_ref[...]
    nb = nb_ref[...]
    onehot = (nb == lax.broadcasted_iota(jnp.int32, (1, _G), 1)).astype(jnp.float32)
    ge = jnp.dot(onehot, gf_ref[...], preferred_element_type=jnp.float32)
    a1 = _sp(jnp.dot(h, w1h_ref[...], preferred_element_type=jnp.float32)
             + jnp.dot(ge, w1g_ref[...], preferred_element_type=jnp.float32)
             + b1_ref[...])
    a = jnp.dot(a1, w2_ref[...], preferred_element_type=jnp.float32) + b2_ref[...]
    amax = jnp.max(a)
    e = jnp.exp(a - amax)
    sg = jnp.dot(onehot.T, e, preferred_element_type=jnp.float32)
    sn = jnp.dot(onehot, sg, preferred_element_type=jnp.float32)
    w = e / (sn + 1e-16)
    hw = h * w
    hg = jnp.dot(onehot.T, hw, preferred_element_type=jnp.float32)
    hg = _sp(jnp.dot(hg, pw_ref[...], preferred_element_type=jnp.float32) + pb_ref[...])
    out = jnp.dot(hg, ow_ref[...], preferred_element_type=jnp.float32) + ob_ref[...]
    out_ref[...] = out


def _final(h, nb2, gf, w1h, w1g, b1, w2, b2, pw, pb, ow, ob):
    return pl.pallas_call(
        _final_body,
        in_specs=[
            pl.BlockSpec((_N, 64), lambda: (0, 0)),
            pl.BlockSpec((_N, 1), lambda: (0, 0)),
            pl.BlockSpec((_G, 103), lambda: (0, 0)),
            pl.BlockSpec((64, 32), lambda: (0, 0)),
            pl.BlockSpec((103, 32), lambda: (0, 0)),
            pl.BlockSpec((1, 32), lambda: (0, 0)),
            pl.BlockSpec((32, 1), lambda: (0, 0)),
            pl.BlockSpec((1, 1), lambda: (0, 0)),
            pl.BlockSpec((64, 64), lambda: (0, 0)),
            pl.BlockSpec((1, 64), lambda: (0, 0)),
            pl.BlockSpec((64, 1), lambda: (0, 0)),
            pl.BlockSpec((1, 1), lambda: (0, 0)),
        ],
        out_specs=pl.BlockSpec((_G, 1), lambda: (0, 0)),
        out_shape=jax.ShapeDtypeStruct((_G, 1), jnp.float32),
        interpret=_INTERPRET,
    )(h, nb2, gf, w1h, w1g, b1, w2, b2, pw, pb, ow, ob)


def _stage0_x_body(x_ref, wx_ref, bx_ref, h_ref):
    h_ref[...] = jnp.dot(x_ref[...], wx_ref[...],
                         preferred_element_type=jnp.float32) + bx_ref[...]


def _stage0_x(x, wx, bx):
    return pl.pallas_call(
        _stage0_x_body,
        in_specs=[
            pl.BlockSpec((_N, 128), lambda: (0, 0)),
            pl.BlockSpec((128, 64), lambda: (0, 0)),
            pl.BlockSpec((1, 64), lambda: (0, 0)),
        ],
        out_specs=pl.BlockSpec((_N, 64), lambda: (0, 0)),
        out_shape=jax.ShapeDtypeStruct((_N, 64), jnp.float32),
        interpret=_INTERPRET,
    )(x, wx, bx)


def _stage0_e_body(ea_ref, we_ref, be_ref, out_ref):
    z = jnp.dot(ea_ref[...], we_ref[...],
                preferred_element_type=jnp.float32) + be_ref[...]
    out_ref[...] = jnp.where(z >= 0, z, 0.2 * z)


def _stage0_e(edge_attr, we, be):
    return pl.pallas_call(
        _stage0_e_body,
        grid=(_NT,),
        in_specs=[
            pl.BlockSpec((_TILE, 16), lambda t: (t, 0)),
            pl.BlockSpec((16, 64), lambda t: (0, 0)),
            pl.BlockSpec((1, 64), lambda t: (0, 0)),
        ],
        out_specs=pl.BlockSpec((_TILE, 64), lambda t: (t, 0)),
        out_shape=jax.ShapeDtypeStruct((_E, 64), jnp.float32),
        interpret=_INTERPRET,
    )(edge_attr, we, be)


def _nodeproj_body(h_ref, wt_ref, gi_ref):
    gi_ref[...] = jnp.dot(h_ref[...], wt_ref[...],
                          preferred_element_type=jnp.float32)


def _passB_body(spr_ref, consts_ref, ev_ref):
    A = consts_ref[0:1, 0:16]
    B = consts_ref[1:2, 0:16]
    c = consts_ref[2:3, 0:16]
    ev = (1.0 + jnp.exp(spr_ref[...] * A + B)) * c
    mask = lax.broadcasted_iota(jnp.int32, (1, 16), 1) < 4
    ev_ref[...] = jnp.where(mask, ev, 0.0)


def _passB(spr, consts):
    return pl.pallas_call(
        _passB_body,
        grid=(_NT,),
        in_specs=[
            pl.BlockSpec((_TILE, 16), lambda t: (t, 0)),
            pl.BlockSpec((4, 128), lambda t: (0, 0)),
        ],
        out_specs=pl.BlockSpec((_TILE, 16), lambda t: (t, 0)),
        out_shape=jax.ShapeDtypeStruct((_E, 16), jnp.float32),
        interpret=_INTERPRET,
    )(spr, consts)


_CH = 100
_NCH = _E // _CH
_CPW = _NCH // _NW
_CPT = _NCH // _NS
_NPS = _N // _NS
_NBUF = 5
_SC_CP = pltpu.CompilerParams(use_tc_tiling_on_sc=False)


def _sc_gather2(h, ii2, ij2):
    @functools.partial(
        pl.kernel,
        out_type=[jax.ShapeDtypeStruct((_E, 64), jnp.float32),
                  jax.ShapeDtypeStruct((_E, 64), jnp.float32)],
        mesh=_SC_MESH,
        compiler_params=_SC_CP,
        scratch_types=[pltpu.VMEM((_CPW, _CH), jnp.int32),
                       pltpu.VMEM((_NBUF, _CH, 64), jnp.float32),
                       pltpu.SemaphoreType.DMA((_NBUF,)),
                       pltpu.SemaphoreType.DMA((_NBUF,))],
    )
    def k(h_hbm, ii_hbm, ij_hbm, xi_hbm, xj_hbm, idx_v, rows_v, gsem, osem):
        wid = lax.axis_index("s") * _NC + lax.axis_index("c")
        cbase = wid * _CPW

        def run(src_hbm, dst_hbm):
            pltpu.sync_copy(src_hbm.at[pl.ds(cbase, _CPW)], idx_v)

            def group(g, _):
                for b in range(_NBUF):
                    i = g * _NBUF + b
                    t = cbase + i

                    @pl.when(g > 0)
                    def _():
                        pltpu.make_async_copy(
                            rows_v.at[b],
                            dst_hbm.at[pl.ds((t - _NBUF) * _CH, _CH)],
                            osem.at[b]).wait()
                    pltpu.async_copy(h_hbm.at[idx_v.at[i]], rows_v.at[b], gsem.at[b])
                for b in range(_NBUF):
                    i = g * _NBUF + b
                    t = cbase + i
                    pltpu.make_async_copy(h_hbm.at[idx_v.at[i]],
                                          rows_v.at[b], gsem.at[b]).wait()
                    pltpu.async_copy(rows_v.at[b],
                                     dst_hbm.at[pl.ds(t * _CH, _CH)], osem.at[b])
                return 0

            ng = _CPW // _NBUF
            lax.fori_loop(0, ng, group, 0)
            for b in range(_NBUF):
                t = cbase + (ng - 1) * _NBUF + b
                pltpu.make_async_copy(rows_v.at[b],
                                      dst_hbm.at[pl.ds(t * _CH, _CH)],
                                      osem.at[b]).wait()

        run(ii_hbm, xi_hbm)
        run(ij_hbm, xj_hbm)

    return k(h, ii2, ij2)


def _sc_softmax_denom(ev16, ii2, zeros16):
    @functools.partial(
        pl.kernel,
        out_type=jax.ShapeDtypeStruct((_E, 16), jnp.float32),
        mesh=_SC_MESH,
        compiler_params=_SC_CP,
        scratch_types=[pltpu.VMEM((_CPT, _CH), jnp.int32),
                       pltpu.VMEM((_NBUF, _CH, 16), jnp.float32),
                       pltpu.VMEM_SHARED((_N, 16), jnp.float32),
                       pltpu.SemaphoreType.DMA((_NBUF,)),
                       pltpu.SemaphoreType.DMA((_NBUF,))],
    )
    def k(ev_hbm, ii_hbm, z_hbm, sv_hbm, idx_v, rows_v, acc_sh, lsem, ssem):
        cid = lax.axis_index("c")
        sid = lax.axis_index("s")
        pltpu.sync_copy(z_hbm.at[pl.ds(sid * _NPS, _NPS)],
                        acc_sh.at[pl.ds(sid * _NPS, _NPS)])
        cbase1 = sid * _CPT
        pltpu.sync_copy(ii_hbm.at[pl.ds(cbase1, _CPT)], idx_v)
        plsc.subcore_barrier()

        def group1(g, _):
            for b in range(_NBUF):
                i = g * _NBUF + b
                t = cbase1 + i

                @pl.when(g > 0)
                def _():
                    pltpu.make_async_copy(rows_v.at[b],
                                          acc_sh.at[idx_v.at[i - _NBUF]],
                                          ssem.at[b]).wait()
                pltpu.async_copy(ev_hbm.at[pl.ds(t * _CH, _CH)],
                                 rows_v.at[b], lsem.at[b])
            for b in range(_NBUF):
                i = g * _NBUF + b
                t = cbase1 + i
                pltpu.make_async_copy(ev_hbm.at[pl.ds(t * _CH, _CH)],
                                      rows_v.at[b], lsem.at[b]).wait()
                pltpu.async_copy(rows_v.at[b], acc_sh.at[idx_v.at[i]],
                                 ssem.at[b], add=True)
            return 0

        ng1 = _CPT // _NBUF
        lax.fori_loop(0, ng1, group1, 0)
        for b in range(_NBUF):
            i = (ng1 - 1) * _NBUF + b
            pltpu.make_async_copy(rows_v.at[b], acc_sh.at[idx_v.at[i]],
                                  ssem.at[b]).wait()
        plsc.subcore_barrier()

        wid = cid * _NS + sid
        cbase2 = wid * _CPW
        pltpu.sync_copy(ii_hbm.at[pl.ds(cbase2, _CPW)], idx_v.at[pl.ds(0, _CPW)])

        def group2(g, _):
            for b in range(_NBUF):
                i = g * _NBUF + b
                t = cbase2 + i

                @pl.when(g > 0)
                def _():
                    pltpu.make_async_copy(
                        rows_v.at[b],
                        sv_hbm.at[pl.ds((t - _NBUF) * _CH, _CH)], ssem.at[b]).wait()
                pltpu.async_copy(acc_sh.at[idx_v.at[i]], rows_v.at[b], lsem.at[b])
            for b in range(_NBUF):
                i = g * _NBUF + b
                t = cbase2 + i
                pltpu.make_async_copy(acc_sh.at[idx_v.at[i]],
                                      rows_v.at[b], lsem.at[b]).wait()
                pltpu.async_copy(rows_v.at[b],
                                 sv_hbm.at[pl.ds(t * _CH, _CH)], ssem.at[b])
            return 0

        ng2 = _CPW // _NBUF
        lax.fori_loop(0, ng2, group2, 0)
        for b in range(_NBUF):
            t = cbase2 + (ng2 - 1) * _NBUF + b
            pltpu.make_async_copy(rows_v.at[b],
                                  sv_hbm.at[pl.ds(t * _CH, _CH)], ssem.at[b]).wait()

    return k(ev16, ii2, zeros16)


def _sc_scatter_m(m, ii2, zeros64):
    @functools.partial(
        pl.kernel,
        out_type=jax.ShapeDtypeStruct((2, _N, 64), jnp.float32),
        mesh=_SC_MESH,
        compiler_params=_SC_CP,
        scratch_types=[pltpu.VMEM((_CPW, _CH), jnp.int32),
                       pltpu.VMEM((_NBUF, _CH, 64), jnp.float32),
                       pltpu.VMEM((_NPS, 64), jnp.float32),
                       pltpu.VMEM_SHARED((_N, 64), jnp.float32),
                       pltpu.SemaphoreType.DMA((_NBUF,)),
                       pltpu.SemaphoreType.DMA((_NBUF,))],
    )
    def k(m_hbm, ii_hbm, z_hbm, out_hbm, idx_v, rows_v, stripe_v, acc_sh,
          lsem, ssem):
        cid = lax.axis_index("c")
        sid = lax.axis_index("s")
        wid = cid * _NS + sid
        cbase = wid * _CPW
        pltpu.sync_copy(z_hbm.at[pl.ds(sid * _NPS, _NPS)],
                        acc_sh.at[pl.ds(sid * _NPS, _NPS)])
        pltpu.sync_copy(ii_hbm.at[pl.ds(cbase, _CPW)], idx_v)
        plsc.subcore_barrier()

        def group(g, _):
            for b in range(_NBUF):
                i = g * _NBUF + b
                t = cbase + i

                @pl.when(g > 0)
                def _():
                    pltpu.make_async_copy(rows_v.at[b],
                                          acc_sh.at[idx_v.at[i - _NBUF]],
                                          ssem.at[b]).wait()
                pltpu.async_copy(m_hbm.at[pl.ds(t * _CH, _CH)],
                                 rows_v.at[b], lsem.at[b])
            for b in range(_NBUF):
                i = g * _NBUF + b
                t = cbase + i
                pltpu.make_async_copy(m_hbm.at[pl.ds(t * _CH, _CH)],
                                      rows_v.at[b], lsem.at[b]).wait()
                pltpu.async_copy(rows_v.at[b], acc_sh.at[idx_v.at[i]],
                                 ssem.at[b], add=True)
            return 0

        ng = _CPW // _NBUF
        lax.fori_loop(0, ng, group, 0)
        for b in range(_NBUF):
            i = (ng - 1) * _NBUF + b
            pltpu.make_async_copy(rows_v.at[b], acc_sh.at[idx_v.at[i]],
                                  ssem.at[b]).wait()
        plsc.subcore_barrier()
        pltpu.sync_copy(acc_sh.at[pl.ds(sid * _NPS, _NPS)], stripe_v)
        pltpu.sync_copy(stripe_v, out_hbm.at[cid, pl.ds(sid * _NPS, _NPS)])

    return k(m, ii2, zeros64)


def kernel(x, edge_source, edge_target, edge_attr, global_fea, node_batch,
           W_x, b_x, W_e, b_e, conv_W, conv_att, conv_bias, bn1_g, bn1_b,
           bn_g, bn_b, ca_W1, ca_b1, ca_W2, ca_b2, post_W, post_b, out_W,
           out_b):
    idx_i = edge_source.astype(jnp.int32)
    idx_j = edge_target.astype(jnp.int32)

    h = _stage0_x(x, W_x, b_x[None, :])
    ea = _stage0_e(edge_attr, W_e, b_e[None, :])

    msel = (jnp.arange(256)[:, None] // 64 == jnp.arange(4)[None, :]).astype(jnp.float32)
    ii2 = idx_i.reshape(_NCH, _CH)
    ij2 = idx_j.reshape(_NCH, _CH)
    zeros16 = jnp.zeros((_N, 16), jnp.float32)
    zeros64 = jnp.zeros((_N, 64), jnp.float32)

    for l in range(3):
        wt = conv_W[l, :64, :]
        wb = conv_W[l, 64:, :]
        z64 = jnp.zeros((64, 256), jnp.float32)
        wbig = jnp.concatenate([
            jnp.concatenate([wt, z64], axis=1),
            jnp.concatenate([z64, wt], axis=1),
            jnp.concatenate([wb, wb], axis=1),
        ], axis=0)
        w2 = jnp.concatenate([wt, wb], axis=0)
        ai_flat = conv_att[l, :, :64].reshape(1, 256)
        aj_flat = conv_att[l, :, 64:].reshape(1, 256)
        gb1 = jnp.stack([
            jnp.pad(bn1_g[l], (0, 124)),
            jnp.pad(bn1_b[l], (0, 124)),
        ])

        xi, xj = _sc_gather2(h, ii2, ij2)
        spr16, consts = _passA(xi, xj, ea, wbig, ai_flat, aj_flat, msel, gb1)
        ev16 = _passB(spr16, consts)
        sv16 = _sc_softmax_denom(ev16, ii2, zeros16)
        m = _passC(xj, ea, ev16, sv16, w2)
        agg = _sc_scatter_m(m, ii2, zeros64)
        gb = jnp.stack([bn_g[l], bn_b[l]])
        h = _passD(agg, conv_bias[l][None, :], gb)

    out = _final(h, node_batch.astype(jnp.int32)[:, None], global_fea,
                 ca_W1[:64, :], ca_W1[64:, :], ca_b1[None, :], ca_W2,
                 ca_b2[None, :], post_W, post_b[None, :], out_W, out_b[None, :])
    return out.reshape(-1)


# speedup vs baseline: 21.9441x; 1.0253x over previous
import functools

import jax
import jax.numpy as jnp
from jax import lax
from jax.experimental import pallas as pl
from jax.experimental.pallas import tpu as pltpu
from jax.experimental.pallas import tpu_sc as plsc

_N, _E, _G = 10000, 160000, 128
_TILE = 5000
_NT = _E // _TILE

_INTERPRET = False

_NC, _NS = 2, 16
_NW = _NC * _NS
_SC_MESH = plsc.VectorSubcoreMesh(core_axis_name="c", subcore_axis_name="s")


def _sp(x):
    return jnp.maximum(x, 0.0) + jnp.log(1.0 + jnp.exp(-jnp.abs(x)))


def _passA_body(xi_ref, xj_ref, ea_ref, wbig_ref, ai_ref, aj_ref,
                msel_ref, gb_ref, spr_ref, consts_ref, acc_ref):
    t = pl.program_id(0)
    xje = jnp.concatenate([xi_ref[...], xj_ref[...], ea_ref[...]], axis=1)
    z = jnp.dot(xje, wbig_ref[...], preferred_element_type=jnp.float32)
    oi = _sp(z[:, 0:256])
    oj = _sp(z[:, 256:512])
    prod = oi * ai_ref[...] + oj * aj_ref[...]
    a_raw = jnp.dot(prod, msel_ref[...], preferred_element_type=jnp.float32)
    spr = _sp(a_raw)
    spr_ref[...] = jnp.concatenate(
        [spr, jnp.zeros((spr.shape[0], 12), jnp.float32)], axis=1)

    pad = jnp.zeros((124,), jnp.float32)
    s1 = jnp.concatenate([jnp.sum(spr, 0), pad])[None, :]
    s2 = jnp.concatenate([jnp.sum(spr * spr, 0), pad])[None, :]
    mn = jnp.concatenate([jnp.min(spr, 0), jnp.full((124,), jnp.inf, jnp.float32)])[None, :]
    mx = jnp.concatenate([jnp.max(spr, 0), jnp.full((124,), -jnp.inf, jnp.float32)])[None, :]

    @pl.when(t == 0)
    def _():
        acc_ref[0:1, :] = s1
        acc_ref[1:2, :] = s2
        acc_ref[2:3, :] = mn
        acc_ref[3:4, :] = mx

    @pl.when(t > 0)
    def _():
        acc_ref[0:1, :] = acc_ref[0:1, :] + s1
        acc_ref[1:2, :] = acc_ref[1:2, :] + s2
        acc_ref[2:3, :] = jnp.minimum(acc_ref[2:3, :], mn)
        acc_ref[3:4, :] = jnp.maximum(acc_ref[3:4, :], mx)

    @pl.when(t == _NT - 1)
    def _():
        g = gb_ref[0:1, :]
        b = gb_ref[1:2, :]
        mu = acc_ref[0:1, :] / _E
        var = acc_ref[1:2, :] / _E - mu * mu
        inv = lax.rsqrt(var + 1e-5)
        A = g * inv
        B = b - g * mu * inv
        y_hi = jnp.maximum(A * acc_ref[3:4, :] + B, A * acc_ref[2:3, :] + B)
        c = 1.0 / (1.0 + jnp.exp(y_hi))
        consts_ref[0:1, :] = A
        consts_ref[1:2, :] = B
        consts_ref[2:3, :] = c


def _passA(xi, xj, ea, wbig, ai_flat, aj_flat, msel, gb):
    return pl.pallas_call(
        _passA_body,
        grid=(_NT,),
        in_specs=[
            pl.BlockSpec((_TILE, 64), lambda t: (t, 0)),
            pl.BlockSpec((_TILE, 64), lambda t: (t, 0)),
            pl.BlockSpec((_TILE, 64), lambda t: (t, 0)),
            pl.BlockSpec((192, 512), lambda t: (0, 0)),
            pl.BlockSpec((1, 256), lambda t: (0, 0)),
            pl.BlockSpec((1, 256), lambda t: (0, 0)),
            pl.BlockSpec((256, 4), lambda t: (0, 0)),
            pl.BlockSpec((2, 128), lambda t: (0, 0)),
        ],
        out_specs=[
            pl.BlockSpec((_TILE, 16), lambda t: (t, 0)),
            pl.BlockSpec((4, 128), lambda t: (0, 0)),
        ],
        out_shape=[
            jax.ShapeDtypeStruct((_E, 16), jnp.float32),
            jax.ShapeDtypeStruct((4, 128), jnp.float32),
        ],
        scratch_shapes=[pltpu.VMEM((4, 128), jnp.float32)],
        interpret=_INTERPRET,
    )(xi, xj, ea, wbig, ai_flat, aj_flat, msel, gb)


def _passC_body(xj_ref, ea_ref, ev_ref, sv_ref, w2_ref, m_ref):
    xe = jnp.concatenate([xj_ref[...], ea_ref[...]], axis=1)
    zj = jnp.dot(xe, w2_ref[...], preferred_element_type=jnp.float32)
    oj = _sp(zj)
    w = ev_ref[:, 0:4] / (sv_ref[:, 0:4] + 1e-16)
    acc = w[:, 0:1] * oj[:, 0:64]
    acc += w[:, 1:2] * oj[:, 64:128]
    acc += w[:, 2:3] * oj[:, 128:192]
    acc += w[:, 3:4] * oj[:, 192:256]
    m_ref[...] = acc * 0.25


def _passC(xj, ea, ev, sv, w2):
    return pl.pallas_call(
        _passC_body,
        grid=(_NT,),
        in_specs=[
            pl.BlockSpec((_TILE, 64), lambda t: (t, 0)),
            pl.BlockSpec((_TILE, 64), lambda t: (t, 0)),
            pl.BlockSpec((_TILE, 16), lambda t: (t, 0)),
            pl.BlockSpec((_TILE, 16), lambda t: (t, 0)),
            pl.BlockSpec((128, 256), lambda t: (0, 0)),
        ],
        out_specs=pl.BlockSpec((_TILE, 64), lambda t: (t, 0)),
        out_shape=jax.ShapeDtypeStruct((_E, 64), jnp.float32),
        interpret=_INTERPRET,
    )(xj, ea, ev, sv, w2)


def _passD_body(agg_ref, bias_ref, gb_ref, h_ref):
    h = agg_ref[0] + agg_ref[1] + bias_ref[...]
    mu = jnp.mean(h, axis=0, keepdims=True)
    var = jnp.mean(h * h, axis=0, keepdims=True) - mu * mu
    inv = lax.rsqrt(var + 1e-5)
    h_ref[...] = gb_ref[0:1, :] * (h - mu) * inv + gb_ref[1:2, :]


def _passD(agg, bias, gb):
    return pl.pallas_call(
        _passD_body,
        in_specs=[
            pl.BlockSpec((2, _N, 64), lambda: (0, 0, 0)),
            pl.BlockSpec((1, 64), lambda: (0, 0)),
            pl.BlockSpec((2, 64), lambda: (0, 0)),
        ],
        out_specs=pl.BlockSpec((_N, 64), lambda: (0, 0)),
        out_shape=jax.ShapeDtypeStruct((_N, 64), jnp.float32),
        interpret=_INTERPRET,
    )(agg, bias, gb)


def _final_body(h_ref, nb_ref, gf_ref, w1h_ref, w1g_ref, b1_ref, w2_ref,
                b2_ref, pw_ref, pb_ref, ow_ref, ob_ref, out_ref):
    h = h_ref[...]
    nb = nb_ref[...]
    onehot = (nb == lax.broadcasted_iota(jnp.int32, (1, _G), 1)).astype(jnp.float32)
    ge = jnp.dot(onehot, gf_ref[...], preferred_element_type=jnp.float32)
    a1 = _sp(jnp.dot(h, w1h_ref[...], preferred_element_type=jnp.float32)
             + jnp.dot(ge, w1g_ref[...], preferred_element_type=jnp.float32)
             + b1_ref[...])
    a = jnp.dot(a1, w2_ref[...], preferred_element_type=jnp.float32) + b2_ref[...]
    amax = jnp.max(a)
    e = jnp.exp(a - amax)
    sg = jnp.dot(onehot.T, e, preferred_element_type=jnp.float32)
    sn = jnp.dot(onehot, sg, preferred_element_type=jnp.float32)
    w = e / (sn + 1e-16)
    hw = h * w
    hg = jnp.dot(onehot.T, hw, preferred_element_type=jnp.float32)
    hg = _sp(jnp.dot(hg, pw_ref[...], preferred_element_type=jnp.float32) + pb_ref[...])
    out = jnp.dot(hg, ow_ref[...], preferred_element_type=jnp.float32) + ob_ref[...]
    out_ref[...] = out


def _final(h, nb2, gf, w1h, w1g, b1, w2, b2, pw, pb, ow, ob):
    return pl.pallas_call(
        _final_body,
        in_specs=[
            pl.BlockSpec((_N, 64), lambda: (0, 0)),
            pl.BlockSpec((_N, 1), lambda: (0, 0)),
            pl.BlockSpec((_G, 103), lambda: (0, 0)),
            pl.BlockSpec((64, 32), lambda: (0, 0)),
            pl.BlockSpec((103, 32), lambda: (0, 0)),
            pl.BlockSpec((1, 32), lambda: (0, 0)),
            pl.BlockSpec((32, 1), lambda: (0, 0)),
            pl.BlockSpec((1, 1), lambda: (0, 0)),
            pl.BlockSpec((64, 64), lambda: (0, 0)),
            pl.BlockSpec((1, 64), lambda: (0, 0)),
            pl.BlockSpec((64, 1), lambda: (0, 0)),
            pl.BlockSpec((1, 1), lambda: (0, 0)),
        ],
        out_specs=pl.BlockSpec((_G, 1), lambda: (0, 0)),
        out_shape=jax.ShapeDtypeStruct((_G, 1), jnp.float32),
        interpret=_INTERPRET,
    )(h, nb2, gf, w1h, w1g, b1, w2, b2, pw, pb, ow, ob)


def _stage0_x_body(x_ref, wx_ref, bx_ref, h_ref):
    h_ref[...] = jnp.dot(x_ref[...], wx_ref[...],
                         preferred_element_type=jnp.float32) + bx_ref[...]


def _stage0_x(x, wx, bx):
    return pl.pallas_call(
        _stage0_x_body,
        in_specs=[
            pl.BlockSpec((_N, 128), lambda: (0, 0)),
            pl.BlockSpec((128, 64), lambda: (0, 0)),
            pl.BlockSpec((1, 64), lambda: (0, 0)),
        ],
        out_specs=pl.BlockSpec((_N, 64), lambda: (0, 0)),
        out_shape=jax.ShapeDtypeStruct((_N, 64), jnp.float32),
        interpret=_INTERPRET,
    )(x, wx, bx)


def _stage0_e_body(ea_ref, we_ref, be_ref, out_ref):
    z = jnp.dot(ea_ref[...], we_ref[...],
                preferred_element_type=jnp.float32) + be_ref[...]
    out_ref[...] = jnp.where(z >= 0, z, 0.2 * z)


def _stage0_e(edge_attr, we, be):
    return pl.pallas_call(
        _stage0_e_body,
        grid=(_NT,),
        in_specs=[
            pl.BlockSpec((_TILE, 16), lambda t: (t, 0)),
            pl.BlockSpec((16, 64), lambda t: (0, 0)),
            pl.BlockSpec((1, 64), lambda t: (0, 0)),
        ],
        out_specs=pl.BlockSpec((_TILE, 64), lambda t: (t, 0)),
        out_shape=jax.ShapeDtypeStruct((_E, 64), jnp.float32),
        interpret=_INTERPRET,
    )(edge_attr, we, be)


def _nodeproj_body(h_ref, wt_ref, gi_ref):
    gi_ref[...] = jnp.dot(h_ref[...], wt_ref[...],
                          preferred_element_type=jnp.float32)


def _passB_body(spr_ref, consts_ref, ev_ref):
    A = consts_ref[0:1, 0:16]
    B = consts_ref[1:2, 0:16]
    c = consts_ref[2:3, 0:16]
    ev = (1.0 + jnp.exp(spr_ref[...] * A + B)) * c
    mask = lax.broadcasted_iota(jnp.int32, (1, 16), 1) < 4
    ev_ref[...] = jnp.where(mask, ev, 0.0)


def _passB(spr, consts):
    return pl.pallas_call(
        _passB_body,
        grid=(_NT,),
        in_specs=[
            pl.BlockSpec((_TILE, 16), lambda t: (t, 0)),
            pl.BlockSpec((4, 128), lambda t: (0, 0)),
        ],
        out_specs=pl.BlockSpec((_TILE, 16), lambda t: (t, 0)),
        out_shape=jax.ShapeDtypeStruct((_E, 16), jnp.float32),
        interpret=_INTERPRET,
    )(spr, consts)


_CH = 100
_NCH = _E // _CH
_CPW = _NCH // _NW
_CPT = _NCH // _NS
_NPS = _N // _NS
_NBUF = 5
_SC_CP = pltpu.CompilerParams(use_tc_tiling_on_sc=False)


def _sc_gather2(h, ii2, ij2):
    @functools.partial(
        pl.kernel,
        out_type=[jax.ShapeDtypeStruct((_E, 64), jnp.float32),
                  jax.ShapeDtypeStruct((_E, 64), jnp.float32)],
        mesh=_SC_MESH,
        compiler_params=_SC_CP,
        scratch_types=[pltpu.VMEM((_CPW, _CH), jnp.int32),
                       pltpu.VMEM((_NBUF, _CH, 64), jnp.float32),
                       pltpu.SemaphoreType.DMA((_NBUF,)),
                       pltpu.SemaphoreType.DMA((_NBUF,))],
    )
    def k(h_hbm, ii_hbm, ij_hbm, xi_hbm, xj_hbm, idx_v, rows_v, gsem, osem):
        wid = lax.axis_index("s") * _NC + lax.axis_index("c")
        cbase = wid * _CPW

        def run(src_hbm, dst_hbm):
            pltpu.sync_copy(src_hbm.at[pl.ds(cbase, _CPW)], idx_v)

            def group(g, _):
                for b in range(_NBUF):
                    i = g * _NBUF + b
                    t = cbase + i

                    @pl.when(g > 0)
                    def _():
                        pltpu.make_async_copy(
                            rows_v.at[b],
                            dst_hbm.at[pl.ds((t - _NBUF) * _CH, _CH)],
                            osem.at[b]).wait()
                    pltpu.async_copy(h_hbm.at[idx_v.at[i]], rows_v.at[b], gsem.at[b])
                for b in range(_NBUF):
                    i = g * _NBUF + b
                    t = cbase + i
                    pltpu.make_async_copy(h_hbm.at[idx_v.at[i]],
                                          rows_v.at[b], gsem.at[b]).wait()
                    pltpu.async_copy(rows_v.at[b],
                                     dst_hbm.at[pl.ds(t * _CH, _CH)], osem.at[b])
                return 0

            ng = _CPW // _NBUF
            lax.fori_loop(0, ng, group, 0)
            for b in range(_NBUF):
                t = cbase + (ng - 1) * _NBUF + b
                pltpu.make_async_copy(rows_v.at[b],
                                      dst_hbm.at[pl.ds(t * _CH, _CH)],
                                      osem.at[b]).wait()

        run(ii_hbm, xi_hbm)
        run(ij_hbm, xj_hbm)

    return k(h, ii2, ij2)


def _sc_softmax_denom(ev16, ii2, zeros16):
    @functools.partial(
        pl.kernel,
        out_type=jax.ShapeDtypeStruct((_E, 16), jnp.float32),
        mesh=_SC_MESH,
        compiler_params=_SC_CP,
        scratch_types=[pltpu.VMEM((_CPT, _CH), jnp.int32),
                       pltpu.VMEM((_NBUF, _CH, 16), jnp.float32),
                       pltpu.VMEM_SHARED((_N, 16), jnp.float32),
                       pltpu.SemaphoreType.DMA((_NBUF,)),
                       pltpu.SemaphoreType.DMA((_NBUF,))],
    )
    def k(ev_hbm, ii_hbm, z_hbm, sv_hbm, idx_v, rows_v, acc_sh, lsem, ssem):
        cid = lax.axis_index("c")
        sid = lax.axis_index("s")
        pltpu.sync_copy(z_hbm.at[pl.ds(sid * _NPS, _NPS)],
                        acc_sh.at[pl.ds(sid * _NPS, _NPS)])
        cbase1 = sid * _CPT
        pltpu.sync_copy(ii_hbm.at[pl.ds(cbase1, _CPT)], idx_v)
        plsc.subcore_barrier()

        def group1(g, _):
            for b in range(_NBUF):
                i = g * _NBUF + b
                t = cbase1 + i

                @pl.when(g > 0)
                def _():
                    pltpu.make_async_copy(rows_v.at[b],
                                          acc_sh.at[idx_v.at[i - _NBUF]],
                                          ssem.at[b]).wait()
                pltpu.async_copy(ev_hbm.at[pl.ds(t * _CH, _CH)],
                                 rows_v.at[b], lsem.at[b])
            for b in range(_NBUF):
                i = g * _NBUF + b
                t = cbase1 + i
                pltpu.make_async_copy(ev_hbm.at[pl.ds(t * _CH, _CH)],
                                      rows_v.at[b], lsem.at[b]).wait()
                pltpu.async_copy(rows_v.at[b], acc_sh.at[idx_v.at[i]],
                                 ssem.at[b], add=True)
            return 0

        ng1 = _CPT // _NBUF
        lax.fori_loop(0, ng1, group1, 0)
        for b in range(_NBUF):
            i = (ng1 - 1) * _NBUF + b
            pltpu.make_async_copy(rows_v.at[b], acc_sh.at[idx_v.at[i]],
                                  ssem.at[b]).wait()
        plsc.subcore_barrier()

        wid = cid * _NS + sid
        cbase2 = wid * _CPW
        pltpu.sync_copy(ii_hbm.at[pl.ds(cbase2, _CPW)], idx_v.at[pl.ds(0, _CPW)])

        def group2(g, _):
            for b in range(_NBUF):
                i = g * _NBUF + b
                t = cbase2 + i

                @pl.when(g > 0)
                def _():
                    pltpu.make_async_copy(
                        rows_v.at[b],
                        sv_hbm.at[pl.ds((t - _NBUF) * _CH, _CH)], ssem.at[b]).wait()
                pltpu.async_copy(acc_sh.at[idx_v.at[i]], rows_v.at[b], lsem.at[b])
            for b in range(_NBUF):
                i = g * _NBUF + b
                t = cbase2 + i
                pltpu.make_async_copy(acc_sh.at[idx_v.at[i]],
                                      rows_v.at[b], lsem.at[b]).wait()
                pltpu.async_copy(rows_v.at[b],
                                 sv_hbm.at[pl.ds(t * _CH, _CH)], ssem.at[b])
            return 0

        ng2 = _CPW // _NBUF
        lax.fori_loop(0, ng2, group2, 0)
        for b in range(_NBUF):
            t = cbase2 + (ng2 - 1) * _NBUF + b
            pltpu.make_async_copy(rows_v.at[b],
                                  sv_hbm.at[pl.ds(t * _CH, _CH)], ssem.at[b]).wait()

    return k(ev16, ii2, zeros16)


def _sc_scatter_m(m, ii2, zeros64):
    @functools.partial(
        pl.kernel,
        out_type=jax.ShapeDtypeStruct((2, _N, 64), jnp.float32),
        mesh=_SC_MESH,
        compiler_params=_SC_CP,
        scratch_types=[pltpu.VMEM((_CPW, _CH), jnp.int32),
                       pltpu.VMEM((_NBUF, _CH, 64), jnp.float32),
                       pltpu.VMEM((_NPS, 64), jnp.float32),
                       pltpu.VMEM_SHARED((_N, 64), jnp.float32),
                       pltpu.SemaphoreType.DMA((_NBUF,)),
                       pltpu.SemaphoreType.DMA((_NBUF,))],
    )
    def k(m_hbm, ii_hbm, z_hbm, out_hbm, idx_v, rows_v, stripe_v, acc_sh,
          lsem, ssem):
        cid = lax.axis_index("c")
        sid = lax.axis_index("s")
        wid = cid * _NS + sid
        cbase = wid * _CPW
        pltpu.sync_copy(z_hbm.at[pl.ds(sid * _NPS, _NPS)],
                        acc_sh.at[pl.ds(sid * _NPS, _NPS)])
        pltpu.sync_copy(ii_hbm.at[pl.ds(cbase, _CPW)], idx_v)
        plsc.subcore_barrier()

        def group(g, _):
            for b in range(_NBUF):
                i = g * _NBUF + b
                t = cbase + i

                @pl.when(g > 0)
                def _():
                    pltpu.make_async_copy(rows_v.at[b],
                                          acc_sh.at[idx_v.at[i - _NBUF]],
                                          ssem.at[b]).wait()
                pltpu.async_copy(m_hbm.at[pl.ds(t * _CH, _CH)],
                                 rows_v.at[b], lsem.at[b])
            for b in range(_NBUF):
                i = g * _NBUF + b
                t = cbase + i
                pltpu.make_async_copy(m_hbm.at[pl.ds(t * _CH, _CH)],
                                      rows_v.at[b], lsem.at[b]).wait()
                pltpu.async_copy(rows_v.at[b], acc_sh.at[idx_v.at[i]],
                                 ssem.at[b], add=True)
            return 0

        ng = _CPW // _NBUF
        lax.fori_loop(0, ng, group, 0)
        for b in range(_NBUF):
            i = (ng - 1) * _NBUF + b
            pltpu.make_async_copy(rows_v.at[b], acc_sh.at[idx_v.at[i]],
                                  ssem.at[b]).wait()
        plsc.subcore_barrier()
        pltpu.sync_copy(acc_sh.at[pl.ds(sid * _NPS, _NPS)], stripe_v)
        pltpu.sync_copy(stripe_v, out_hbm.at[cid, pl.ds(sid * _NPS, _NPS)])

    return k(m, ii2, zeros64)


def kernel(x, edge_source, edge_target, edge_attr, global_fea, node_batch,
           W_x, b_x, W_e, b_e, conv_W, conv_att, conv_bias, bn1_g, bn1_b,
           bn_g, bn_b, ca_W1, ca_b1, ca_W2, ca_b2, post_W, post_b, out_W,
           out_b):
    idx_i = edge_source.astype(jnp.int32)
    idx_j = edge_target.astype(jnp.int32)

    h = _stage0_x(x, W_x, b_x[None, :])
    ea = _stage0_e(edge_attr, W_e, b_e[None, :])

    msel = (jnp.arange(256)[:, None] // 64 == jnp.arange(4)[None, :]).astype(jnp.float32)
    ii2 = idx_i.reshape(_NCH, _CH)
    ij2 = idx_j.reshape(_NCH, _CH)
    zeros16 = jnp.zeros((_N, 16), jnp.float32)
    zeros64 = jnp.zeros((_N, 64), jnp.float32)

    for l in range(3):
        wt = conv_W[l, :64, :]
        wb = conv_W[l, 64:, :]
        z64 = jnp.zeros((64, 256), jnp.float32)
        wbig = jnp.concatenate([
            jnp.concatenate([wt, z64], axis=1),
            jnp.concatenate([z64, wt], axis=1),
            jnp.concatenate([wb, wb], axis=1),
        ], axis=0)
        w2 = jnp.concatenate([wt, wb], axis=0)
        ai_flat = conv_att[l, :, :64].reshape(1, 256)
        aj_flat = conv_att[l, :, 64:].reshape(1, 256)
        gb1 = jnp.stack([
            jnp.pad(bn1_g[l], (0, 124)),
            jnp.pad(bn1_b[l], (0, 124)),
        ])

        xi, xj = _sc_gather2(h, ii2, ij2)
        spr16, consts = _passA(xi, xj, ea, wbig, ai_flat, aj_flat, msel, gb1)
        ev16 = _passB(spr16, consts)
        sv16 = _sc_softmax_denom(ev16, ii2, zeros16)
        m = _passC(xj, ea, ev16, sv16, w2)
        agg = _sc_scatter_m(m, ii2, zeros64)
        gb = jnp.stack([bn_g[l], bn_b[l]])
        h = _passD(agg, conv_bias[l][None, :], gb)

    out = _final(h, node_batch.astype(jnp.int32)[:, None], global_fea,
                 ca_W1[:64, :], ca_W1[64:, :], ca_b1[None, :], ca_W2,
                 ca_b2[None, :], post_W, post_b[None, :], out_W, out_b[None, :])
    return out.reshape(-1)
